# Initial kernel scaffold; baseline (speedup 1.0000x reference)
import functools

import jax
import jax.numpy as jnp
from jax import lax
from jax.experimental import pallas as pl
from jax.experimental.pallas import tpu as pltpu
from jax.experimental.pallas import tpu_sc as plsc

N = 100000
E = 3200000
B = 128
D = 16
VOCAB = 1032

NS = 16
CH = 128
NP = 102400
EP = 3276800
NCH_N = NP // (NS * CH)
NCH_E = EP // (NS * CH)
GRP = 32
ROWS_PER_TILE = NP // NS
BLK = 1024
NBLK = NP // BLK

_mesh = plsc.VectorSubcoreMesh(core_axis_name="c", subcore_axis_name="s")

_f32 = jnp.float32
_i32 = jnp.int32


@functools.partial(
    pl.kernel,
    out_type=(
        jax.ShapeDtypeStruct((NP, D), _f32),
        jax.ShapeDtypeStruct((NP, D), _f32),
        jax.ShapeDtypeStruct((NP, D), _f32),
        jax.ShapeDtypeStruct((NP, D), _f32),
    ),
    mesh=_mesh,
    scratch_types=[
        pltpu.VMEM_SHARED((NP, D), _f32),
        pltpu.VMEM((NCH_N, CH), _i32),
        pltpu.VMEM((GRP, CH), _i32),
        pltpu.VMEM((CH, D), _f32),
        pltpu.VMEM((CH, D), _f32),
    ],
)
def _sc_emb_deg(x_r, x_l, dst_r, dst_l, emb, zeros, ones,
                h0_r, h0_l, deg_r, deg_l,
                acc, xbuf, ibuf, rows, ones_v):
    c = lax.axis_index("c")
    t = lax.axis_index("s")
    sl = pl.ds(t * ROWS_PER_TILE, ROWS_PER_TILE)

    def side(x_hbm, dst_hbm, h0_hbm, deg_hbm):
        pltpu.sync_copy(zeros.at[sl], acc.at[sl])
        pltpu.sync_copy(ones, ones_v)
        pltpu.sync_copy(x_hbm.at[pl.ds(t * NCH_N, NCH_N)], xbuf)

        @pl.loop(0, NCH_N)
        def _(k):
            pltpu.sync_copy(emb.at[xbuf.at[k]], rows)
            pltpu.sync_copy(rows, h0_hbm.at[pl.ds((t * NCH_N + k) * CH, CH)])

        plsc.subcore_barrier()

        @pl.loop(0, NCH_E // GRP)
        def _(g):
            pltpu.sync_copy(dst_hbm.at[pl.ds(t * NCH_E + g * GRP, GRP)], ibuf)

            @pl.loop(0, GRP)
            def _(j):
                pltpu.sync_copy(ones_v, acc.at[ibuf.at[j]], add=True)

        plsc.subcore_barrier()
        pltpu.sync_copy(acc.at[sl], deg_hbm.at[sl])

    @pl.when(c == 0)
    def _():
        side(x_r, dst_r, h0_r, deg_r)

    @pl.when(c == 1)
    def _():
        side(x_l, dst_l, h0_l, deg_l)


@functools.partial(
    pl.kernel,
    out_type=(
        jax.ShapeDtypeStruct((NP, D), _f32),
        jax.ShapeDtypeStruct((NP, D), _f32),
    ),
    mesh=_mesh,
    scratch_types=[
        pltpu.VMEM_SHARED((NP, D), _f32),
        pltpu.VMEM((GRP, CH), _i32),
        pltpu.VMEM((GRP, CH), _i32),
        pltpu.VMEM((CH, D), _f32),
    ],
)
def _sc_aggregate(y_r, y_l, src_r, src_l, dst_r, dst_l, zeros,
                  out_r, out_l,
                  acc, sbuf, dbuf, rows):
    c = lax.axis_index("c")
    t = lax.axis_index("s")
    sl = pl.ds(t * ROWS_PER_TILE, ROWS_PER_TILE)

    def side(y_hbm, src_hbm, dst_hbm, out_hbm):
        pltpu.sync_copy(zeros.at[sl], acc.at[sl])
        plsc.subcore_barrier()

        @pl.loop(0, NCH_E // GRP)
        def _(g):
            base = t * NCH_E + g * GRP
            pltpu.sync_copy(src_hbm.at[pl.ds(base, GRP)], sbuf)
            pltpu.sync_copy(dst_hbm.at[pl.ds(base, GRP)], dbuf)

            @pl.loop(0, GRP)
            def _(j):
                pltpu.sync_copy(y_hbm.at[sbuf.at[j]], rows)
                pltpu.sync_copy(rows, acc.at[dbuf.at[j]], add=True)

        plsc.subcore_barrier()
        pltpu.sync_copy(acc.at[sl], out_hbm.at[sl])

    @pl.when(c == 0)
    def _():
        side(y_r, src_r, dst_r, out_r)

    @pl.when(c == 1)
    def _():
        side(y_l, src_l, dst_l, out_l)


def _tc_y1(h0, deg, W1):
    def body(h_ref, d_ref, w_ref, y_ref):
        dinv = lax.rsqrt(d_ref[:, 0:1] + 1.0)
        y_ref[...] = jnp.dot(h_ref[...], w_ref[...],
                             preferred_element_type=_f32) * dinv

    return pl.pallas_call(
        body,
        grid=(NBLK,),
        in_specs=[
            pl.BlockSpec((BLK, D), lambda i: (i, 0)),
            pl.BlockSpec((BLK, D), lambda i: (i, 0)),
            pl.BlockSpec((D, D), lambda i: (0, 0)),
        ],
        out_specs=pl.BlockSpec((BLK, D), lambda i: (i, 0)),
        out_shape=jax.ShapeDtypeStruct((NP, D), _f32),
    )(h0, deg, W1)


def _tc_y2(acc1, y1, deg, W2, b1b):
    def body(a_ref, y_ref, d_ref, w_ref, b_ref, o_ref):
        dinv = lax.rsqrt(d_ref[:, 0:1] + 1.0)
        h1 = (a_ref[...] + y_ref[...]) * dinv + b_ref[0:1, :]
        h1 = jnp.maximum(h1, 0.0)
        o_ref[...] = jnp.dot(h1, w_ref[...], preferred_element_type=_f32) * dinv

    return pl.pallas_call(
        body,
        grid=(NBLK,),
        in_specs=[
            pl.BlockSpec((BLK, D), lambda i: (i, 0)),
            pl.BlockSpec((BLK, D), lambda i: (i, 0)),
            pl.BlockSpec((BLK, D), lambda i: (i, 0)),
            pl.BlockSpec((D, D), lambda i: (0, 0)),
            pl.BlockSpec((8, D), lambda i: (0, 0)),
        ],
        out_specs=pl.BlockSpec((BLK, D), lambda i: (i, 0)),
        out_shape=jax.ShapeDtypeStruct((NP, D), _f32),
    )(acc1, y1, deg, W2, b1b)


def _tc_epilogue(acc_r, y_r, deg_r, batch_r, acc_l, y_l, deg_l, batch_l,
                 b2b, Wfc8, bfc8):
    def body(ar, yr, dr, br, al, yl, dl, bl, b2_ref, w_ref, bf_ref, o_ref,
             s_r, c_r, s_l, c_l):
        i = pl.program_id(0)

        @pl.when(i == 0)
        def _():
            s_r[...] = jnp.zeros_like(s_r)
            c_r[...] = jnp.zeros_like(c_r)
            s_l[...] = jnp.zeros_like(s_l)
            c_l[...] = jnp.zeros_like(c_l)

        def side(a_ref, y_ref, d_ref, b_ref, s_scr, c_scr):
            dinv = lax.rsqrt(d_ref[:, 0:1] + 1.0)
            h2 = (a_ref[...] + y_ref[...]) * dinv + b2_ref[0:1, :]
            bid = b_ref[0]
            oh = (lax.broadcasted_iota(_i32, (B, BLK), 0) == bid).astype(_f32)
            s_scr[...] += jnp.dot(oh, h2, preferred_element_type=_f32)
            c_scr[...] += jnp.sum(oh, axis=1, keepdims=True)

        side(ar, yr, dr, br, s_r, c_r)
        side(al, yl, dl, bl, s_l, c_l)

        @pl.when(i == NBLK - 1)
        def _():
            mr = s_r[...] / jnp.maximum(c_r[...], 1.0)
            ml = s_l[...] / jnp.maximum(c_l[...], 1.0)
            hcat = jnp.concatenate([mr, ml], axis=1)
            out = lax.dot_general(hcat, w_ref[...],
                                  (((1,), (1,)), ((), ())),
                                  preferred_element_type=_f32)
            o_ref[...] = out + bf_ref[0:1, :]

    node_spec = pl.BlockSpec((BLK, D), lambda i: (i, 0))
    batch_spec = pl.BlockSpec((1, 1, BLK), lambda i: (i, 0, 0))
    return pl.pallas_call(
        body,
        grid=(NBLK,),
        in_specs=[
            node_spec, node_spec, node_spec, batch_spec,
            node_spec, node_spec, node_spec, batch_spec,
            pl.BlockSpec((8, D), lambda i: (0, 0)),
            pl.BlockSpec((8, 2 * D), lambda i: (0, 0)),
            pl.BlockSpec((8, 8), lambda i: (0, 0)),
        ],
        out_specs=pl.BlockSpec((B, 8), lambda i: (0, 0)),
        out_shape=jax.ShapeDtypeStruct((B, 8), _f32),
        scratch_shapes=[
            pltpu.VMEM((B, D), _f32),
            pltpu.VMEM((B, 1), _f32),
            pltpu.VMEM((B, D), _f32),
            pltpu.VMEM((B, 1), _f32),
        ],
    )(acc_r, y_r, deg_r, batch_r, acc_l, y_l, deg_l, batch_l, b2b, Wfc8, bfc8)


def _prep_nodes(x):
    xp = jnp.concatenate([x.astype(_i32), jnp.zeros((NP - N,), _i32)])
    return xp.reshape(NP // CH, CH)


def _prep_edges(ei):
    src = jnp.concatenate([ei[0].astype(_i32), jnp.zeros((EP - E,), _i32)])
    dst = jnp.concatenate([ei[1].astype(_i32), jnp.full((EP - E,), N, _i32)])
    return src.reshape(EP // CH, CH), dst.reshape(EP // CH, CH)


def _prep_batch(b):
    bp = jnp.concatenate([b.astype(_i32), jnp.full((NP - N,), B + 7, _i32)])
    return bp.reshape(NBLK, 1, BLK)


def kernel(receptor_x, receptor_edge_index, receptor_batch,
           ligand_x, ligand_edge_index, ligand_batch,
           emb_table, W1, b1, W2, b2, Wfc, bfc):
    x_r = _prep_nodes(receptor_x)
    x_l = _prep_nodes(ligand_x)
    src_r, dst_r = _prep_edges(receptor_edge_index)
    src_l, dst_l = _prep_edges(ligand_edge_index)
    batch_r = _prep_batch(receptor_batch)
    batch_l = _prep_batch(ligand_batch)

    zeros = jnp.zeros((NP, D), _f32)
    ones = jnp.ones((CH, D), _f32)
    b1b = jnp.tile(b1[None, :], (8, 1))
    b2b = jnp.tile(b2[None, :], (8, 1))
    Wfc8 = jnp.concatenate([Wfc, jnp.zeros((2, 2 * D), _f32)], axis=0)
    bfc8 = jnp.tile(jnp.concatenate([bfc, jnp.zeros((2,), _f32)])[None, :],
                    (8, 1))

    h0_r, h0_l, deg_r, deg_l = _sc_emb_deg(
        x_r, x_l, dst_r, dst_l, emb_table, zeros, ones)

    y1_r = _tc_y1(h0_r, deg_r, W1)
    y1_l = _tc_y1(h0_l, deg_l, W1)

    acc1_r, acc1_l = _sc_aggregate(y1_r, y1_l, src_r, src_l, dst_r, dst_l,
                                   zeros)

    y2_r = _tc_y2(acc1_r, y1_r, deg_r, W2, b1b)
    y2_l = _tc_y2(acc1_l, y1_l, deg_l, W2, b1b)

    acc2_r, acc2_l = _sc_aggregate(y2_r, y2_l, src_r, src_l, dst_r, dst_l,
                                   zeros)

    out8 = _tc_epilogue(acc2_r, y2_r, deg_r, batch_r,
                        acc2_l, y2_l, deg_l, batch_l,
                        b2b, Wfc8, bfc8)
    return (out8[:, :3], out8[:, 3:6])


# speedup vs baseline: 39.9483x; 39.9483x over previous
import functools

import jax
import jax.numpy as jnp
from jax import lax
from jax.experimental import pallas as pl
from jax.experimental.pallas import tpu as pltpu
from jax.experimental.pallas import tpu_sc as plsc

N = 100000
E = 3200000
B = 128
D = 16
VOCAB = 1032

NS = 16
CH = 128
NP = 102400
EP = 3276800
NCH_N = NP // (NS * CH)
NCH_E = EP // (NS * CH)
GRP = 32
ROWS_PER_TILE = NP // NS
BLK = 1024
NBLK = NP // BLK

_f32 = jnp.float32
_i32 = jnp.int32

_SC_PARAMS = pltpu.CompilerParams(use_tc_tiling_on_sc=False)


@functools.cache
def _build_sc_emb_deg():
    @functools.partial(
        pl.kernel,
        out_type=(
            jax.ShapeDtypeStruct((NP, D), _f32),
            jax.ShapeDtypeStruct((NP, D), _f32),
            jax.ShapeDtypeStruct((NP, D), _f32),
            jax.ShapeDtypeStruct((NP, D), _f32),
        ),
        mesh=plsc.VectorSubcoreMesh(core_axis_name="c", subcore_axis_name="s"),
        compiler_params=_SC_PARAMS,
        scratch_types=[
            pltpu.VMEM_SHARED((NP, D), _f32),
            pltpu.VMEM((NCH_N, CH), _i32),
            pltpu.VMEM((GRP, CH), _i32),
            pltpu.VMEM((CH, D), _f32),
            pltpu.VMEM((CH, D), _f32),
        ],
    )
    def sc_emb_deg(x_r, x_l, dst_r, dst_l, emb, zeros, ones,
                   h0_r, h0_l, deg_r, deg_l,
                   acc, xbuf, ibuf, rows, ones_v):
        c = lax.axis_index("c")
        t = lax.axis_index("s")
        sl = pl.ds(t * ROWS_PER_TILE, ROWS_PER_TILE)

        def side(x_hbm, dst_hbm, h0_hbm, deg_hbm):
            pltpu.sync_copy(zeros.at[sl], acc.at[sl])
            pltpu.sync_copy(ones, ones_v)
            pltpu.sync_copy(x_hbm.at[t], xbuf)

            @pl.loop(0, NCH_N)
            def _(k):
                pltpu.sync_copy(emb.at[xbuf.at[k]], rows)
                pltpu.sync_copy(rows,
                                h0_hbm.at[pl.ds((t * NCH_N + k) * CH, CH)])

            plsc.subcore_barrier()

            @pl.loop(0, NCH_E // GRP)
            def _(g):
                pltpu.sync_copy(dst_hbm.at[pl.ds(t * NCH_E + g * GRP, GRP)],
                                ibuf)

                @pl.loop(0, GRP)
                def _(j):
                    pltpu.sync_copy(ones_v, acc.at[ibuf.at[j]], add=True)

            plsc.subcore_barrier()
            pltpu.sync_copy(acc.at[sl], deg_hbm.at[sl])

        @pl.when(c == 0)
        def _():
            side(x_r, dst_r, h0_r, deg_r)

        @pl.when(c == 1)
        def _():
            side(x_l, dst_l, h0_l, deg_l)

    return sc_emb_deg


@functools.cache
def _build_sc_aggregate():
    @functools.partial(
        pl.kernel,
        out_type=(
            jax.ShapeDtypeStruct((NP, D), _f32),
            jax.ShapeDtypeStruct((NP, D), _f32),
        ),
        mesh=plsc.VectorSubcoreMesh(core_axis_name="c", subcore_axis_name="s"),
        compiler_params=_SC_PARAMS,
        scratch_types=[
            pltpu.VMEM_SHARED((NP, D), _f32),
            pltpu.VMEM((GRP, CH), _i32),
            pltpu.VMEM((GRP, CH), _i32),
            pltpu.VMEM((CH, D), _f32),
        ],
    )
    def sc_aggregate(y_r, y_l, src_r, src_l, dst_r, dst_l, zeros,
                     out_r, out_l,
                     acc, sbuf, dbuf, rows):
        c = lax.axis_index("c")
        t = lax.axis_index("s")
        sl = pl.ds(t * ROWS_PER_TILE, ROWS_PER_TILE)

        def side(y_hbm, src_hbm, dst_hbm, out_hbm):
            pltpu.sync_copy(zeros.at[sl], acc.at[sl])
            plsc.subcore_barrier()

            @pl.loop(0, NCH_E // GRP)
            def _(g):
                base = t * NCH_E + g * GRP
                pltpu.sync_copy(src_hbm.at[pl.ds(base, GRP)], sbuf)
                pltpu.sync_copy(dst_hbm.at[pl.ds(base, GRP)], dbuf)

                @pl.loop(0, GRP)
                def _(j):
                    pltpu.sync_copy(y_hbm.at[sbuf.at[j]], rows)
                    pltpu.sync_copy(rows, acc.at[dbuf.at[j]], add=True)

            plsc.subcore_barrier()
            pltpu.sync_copy(acc.at[sl], out_hbm.at[sl])

        @pl.when(c == 0)
        def _():
            side(y_r, src_r, dst_r, out_r)

        @pl.when(c == 1)
        def _():
            side(y_l, src_l, dst_l, out_l)

    return sc_aggregate


def _tc_y1(h0, deg, W1):
    def body(h_ref, d_ref, w_ref, y_ref):
        dinv = lax.rsqrt(d_ref[:, 0:1] + 1.0)
        y_ref[...] = jnp.dot(h_ref[...], w_ref[...],
                             preferred_element_type=_f32) * dinv

    return pl.pallas_call(
        body,
        grid=(NBLK,),
        in_specs=[
            pl.BlockSpec((BLK, D), lambda i: (i, 0)),
            pl.BlockSpec((BLK, D), lambda i: (i, 0)),
            pl.BlockSpec((D, D), lambda i: (0, 0)),
        ],
        out_specs=pl.BlockSpec((BLK, D), lambda i: (i, 0)),
        out_shape=jax.ShapeDtypeStruct((NP, D), _f32),
    )(h0, deg, W1)


def _tc_y2(acc1, y1, deg, W2, b1b):
    def body(a_ref, y_ref, d_ref, w_ref, b_ref, o_ref):
        dinv = lax.rsqrt(d_ref[:, 0:1] + 1.0)
        h1 = (a_ref[...] + y_ref[...]) * dinv + b_ref[0:1, :]
        h1 = jnp.maximum(h1, 0.0)
        o_ref[...] = jnp.dot(h1, w_ref[...], preferred_element_type=_f32) * dinv

    return pl.pallas_call(
        body,
        grid=(NBLK,),
        in_specs=[
            pl.BlockSpec((BLK, D), lambda i: (i, 0)),
            pl.BlockSpec((BLK, D), lambda i: (i, 0)),
            pl.BlockSpec((BLK, D), lambda i: (i, 0)),
            pl.BlockSpec((D, D), lambda i: (0, 0)),
            pl.BlockSpec((8, D), lambda i: (0, 0)),
        ],
        out_specs=pl.BlockSpec((BLK, D), lambda i: (i, 0)),
        out_shape=jax.ShapeDtypeStruct((NP, D), _f32),
    )(acc1, y1, deg, W2, b1b)


def _tc_epilogue(acc_r, y_r, deg_r, batch_r, acc_l, y_l, deg_l, batch_l,
                 b2b, Wfc8, bfc8):
    def body(ar, yr, dr, br, al, yl, dl, bl, b2_ref, w_ref, bf_ref, o_ref,
             s_r, c_r, s_l, c_l):
        i = pl.program_id(0)

        @pl.when(i == 0)
        def _():
            s_r[...] = jnp.zeros_like(s_r)
            c_r[...] = jnp.zeros_like(c_r)
            s_l[...] = jnp.zeros_like(s_l)
            c_l[...] = jnp.zeros_like(c_l)

        def side(a_ref, y_ref, d_ref, b_ref, s_scr, c_scr):
            dinv = lax.rsqrt(d_ref[:, 0:1] + 1.0)
            h2 = (a_ref[...] + y_ref[...]) * dinv + b2_ref[0:1, :]
            bid = b_ref[0]
            oh = (lax.broadcasted_iota(_i32, (B, BLK), 0) == bid).astype(_f32)
            s_scr[...] += jnp.dot(oh, h2, preferred_element_type=_f32)
            c_scr[...] += jnp.sum(oh, axis=1, keepdims=True)

        side(ar, yr, dr, br, s_r, c_r)
        side(al, yl, dl, bl, s_l, c_l)

        @pl.when(i == NBLK - 1)
        def _():
            mr = s_r[...] / jnp.maximum(c_r[...], 1.0)
            ml = s_l[...] / jnp.maximum(c_l[...], 1.0)
            hcat = jnp.concatenate([mr, ml], axis=1)
            out = lax.dot_general(hcat, w_ref[...],
                                  (((1,), (1,)), ((), ())),
                                  preferred_element_type=_f32)
            o_ref[...] = out + bf_ref[0:1, :]

    node_spec = pl.BlockSpec((BLK, D), lambda i: (i, 0))
    batch_spec = pl.BlockSpec((1, 1, BLK), lambda i: (i, 0, 0))
    return pl.pallas_call(
        body,
        grid=(NBLK,),
        in_specs=[
            node_spec, node_spec, node_spec, batch_spec,
            node_spec, node_spec, node_spec, batch_spec,
            pl.BlockSpec((8, D), lambda i: (0, 0)),
            pl.BlockSpec((8, 2 * D), lambda i: (0, 0)),
            pl.BlockSpec((8, 8), lambda i: (0, 0)),
        ],
        out_specs=pl.BlockSpec((B, 8), lambda i: (0, 0)),
        out_shape=jax.ShapeDtypeStruct((B, 8), _f32),
        scratch_shapes=[
            pltpu.VMEM((B, D), _f32),
            pltpu.VMEM((B, 1), _f32),
            pltpu.VMEM((B, D), _f32),
            pltpu.VMEM((B, 1), _f32),
        ],
    )(acc_r, y_r, deg_r, batch_r, acc_l, y_l, deg_l, batch_l, b2b, Wfc8, bfc8)


def _prep_nodes(x):
    xp = jnp.concatenate([x.astype(_i32), jnp.zeros((NP - N,), _i32)])
    return xp.reshape(NS, NCH_N, CH)


def _prep_edges(ei):
    src = jnp.concatenate([ei[0].astype(_i32), jnp.zeros((EP - E,), _i32)])
    dst = jnp.concatenate([ei[1].astype(_i32), jnp.full((EP - E,), N, _i32)])
    return src.reshape(EP // CH, CH), dst.reshape(EP // CH, CH)


def _prep_batch(b):
    bp = jnp.concatenate([b.astype(_i32), jnp.full((NP - N,), B + 7, _i32)])
    return bp.reshape(NBLK, 1, BLK)


def kernel(receptor_x, receptor_edge_index, receptor_batch,
           ligand_x, ligand_edge_index, ligand_batch,
           emb_table, W1, b1, W2, b2, Wfc, bfc):
    x_r = _prep_nodes(receptor_x)
    x_l = _prep_nodes(ligand_x)
    src_r, dst_r = _prep_edges(receptor_edge_index)
    src_l, dst_l = _prep_edges(ligand_edge_index)
    batch_r = _prep_batch(receptor_batch)
    batch_l = _prep_batch(ligand_batch)

    zeros = jnp.zeros((NP, D), _f32)
    ones = jnp.ones((CH, D), _f32)
    b1b = jnp.tile(b1[None, :], (8, 1))
    b2b = jnp.tile(b2[None, :], (8, 1))
    Wfc8 = jnp.concatenate([Wfc, jnp.zeros((2, 2 * D), _f32)], axis=0)
    bfc8 = jnp.tile(jnp.concatenate([bfc, jnp.zeros((2,), _f32)])[None, :],
                    (8, 1))

    h0_r, h0_l, deg_r, deg_l = _build_sc_emb_deg()(
        x_r, x_l, dst_r, dst_l, emb_table, zeros, ones)

    y1_r = _tc_y1(h0_r, deg_r, W1)
    y1_l = _tc_y1(h0_l, deg_l, W1)

    acc1_r, acc1_l = _build_sc_aggregate()(
        y1_r, y1_l, src_r, src_l, dst_r, dst_l, zeros)

    y2_r = _tc_y2(acc1_r, y1_r, deg_r, W2, b1b)
    y2_l = _tc_y2(acc1_l, y1_l, deg_l, W2, b1b)

    acc2_r, acc2_l = _build_sc_aggregate()(
        y2_r, y2_l, src_r, src_l, dst_r, dst_l, zeros)

    out8 = _tc_epilogue(acc2_r, y2_r, deg_r, batch_r,
                        acc2_l, y2_l, deg_l, batch_l,
                        b2b, Wfc8, bfc8)
    return (out8[:, :3], out8[:, 3:6])


# speedup vs baseline: 65.2545x; 1.6335x over previous
import functools

import jax
import jax.numpy as jnp
from jax import lax
from jax.experimental import pallas as pl
from jax.experimental.pallas import tpu as pltpu
from jax.experimental.pallas import tpu_sc as plsc

N = 100000
E = 3200000
B = 128
D = 16
VOCAB = 1032

NS = 16
CH = 128
NP = 102400
EP = 3276800
NCH_N = NP // (NS * CH)
NCH_E = EP // (NS * CH)
GRP = 32
AGRP = 32
NBUF = 4
ROWS_PER_TILE = NP // NS
BLK = 1024
NBLK = NP // BLK

_f32 = jnp.float32
_i32 = jnp.int32

_SC_PARAMS = pltpu.CompilerParams(use_tc_tiling_on_sc=False)


@functools.cache
def _build_sc_emb_deg():
    @functools.partial(
        pl.kernel,
        out_type=(
            jax.ShapeDtypeStruct((NP, D), _f32),
            jax.ShapeDtypeStruct((NP, D), _f32),
            jax.ShapeDtypeStruct((NP, D), _f32),
            jax.ShapeDtypeStruct((NP, D), _f32),
        ),
        mesh=plsc.VectorSubcoreMesh(core_axis_name="c", subcore_axis_name="s"),
        compiler_params=_SC_PARAMS,
        scratch_types=[
            pltpu.VMEM_SHARED((NP, D), _f32),
            pltpu.VMEM((NCH_N, CH), _i32),
            pltpu.VMEM((GRP, CH), _i32),
            pltpu.VMEM((CH, D), _f32),
            pltpu.VMEM((CH, D), _f32),
        ],
    )
    def sc_emb_deg(x_r, x_l, dst_r, dst_l, emb, zeros, ones,
                   h0_r, h0_l, deg_r, deg_l,
                   acc, xbuf, ibuf, rows, ones_v):
        c = lax.axis_index("c")
        t = lax.axis_index("s")
        sl = pl.ds(t * ROWS_PER_TILE, ROWS_PER_TILE)

        def side(x_hbm, dst_hbm, h0_hbm, deg_hbm):
            pltpu.sync_copy(zeros.at[sl], acc.at[sl])
            pltpu.sync_copy(ones, ones_v)
            pltpu.sync_copy(x_hbm.at[t], xbuf)

            @pl.loop(0, NCH_N)
            def _(k):
                pltpu.sync_copy(emb.at[xbuf.at[k]], rows)
                pltpu.sync_copy(rows,
                                h0_hbm.at[pl.ds((t * NCH_N + k) * CH, CH)])

            plsc.subcore_barrier()

            @pl.loop(0, NCH_E // GRP)
            def _(g):
                pltpu.sync_copy(dst_hbm.at[pl.ds(t * NCH_E + g * GRP, GRP)],
                                ibuf)

                @pl.loop(0, GRP)
                def _(j):
                    pltpu.sync_copy(ones_v, acc.at[ibuf.at[j]], add=True)

            plsc.subcore_barrier()
            pltpu.sync_copy(acc.at[sl], deg_hbm.at[sl])

        @pl.when(c == 0)
        def _():
            side(x_r, dst_r, h0_r, deg_r)

        @pl.when(c == 1)
        def _():
            side(x_l, dst_l, h0_l, deg_l)

    return sc_emb_deg


@functools.cache
def _build_sc_aggregate():
    @functools.partial(
        pl.kernel,
        out_type=(
            jax.ShapeDtypeStruct((NP, D), _f32),
            jax.ShapeDtypeStruct((NP, D), _f32),
        ),
        mesh=plsc.VectorSubcoreMesh(core_axis_name="c", subcore_axis_name="s"),
        compiler_params=_SC_PARAMS,
        scratch_types=[
            pltpu.VMEM_SHARED((NP, D), _f32),
            pltpu.VMEM((2, AGRP, CH), _i32),
            pltpu.VMEM((2, AGRP, CH), _i32),
            pltpu.VMEM((NBUF, CH, D), _f32),
            pltpu.SemaphoreType.DMA((NBUF,)),
            pltpu.SemaphoreType.DMA,
        ],
    )
    def sc_aggregate(y_r, y_l, src_r, src_l, dst_r, dst_l, zeros,
                     out_r, out_l,
                     acc, sbuf, dbuf, rows, gsem, ssem):
        c = lax.axis_index("c")
        t = lax.axis_index("s")
        sl = pl.ds(t * ROWS_PER_TILE, ROWS_PER_TILE)
        ngrp = NCH_E // AGRP

        def side(y_hbm, src_hbm, dst_hbm, out_hbm):
            pltpu.sync_copy(zeros.at[sl], acc.at[sl])
            plsc.subcore_barrier()

            def stage_start(g, slot):
                base = t * NCH_E + g * AGRP
                pltpu.make_async_copy(src_hbm.at[pl.ds(base, AGRP)],
                                      sbuf.at[slot], ssem).start()
                pltpu.make_async_copy(dst_hbm.at[pl.ds(base, AGRP)],
                                      dbuf.at[slot], ssem).start()

            def stage_wait(slot):
                pltpu.make_async_copy(src_hbm.at[pl.ds(t * NCH_E, AGRP)],
                                      sbuf.at[slot], ssem).wait()
                pltpu.make_async_copy(dst_hbm.at[pl.ds(t * NCH_E, AGRP)],
                                      dbuf.at[slot], ssem).wait()

            def gather_start(slot, j, b):
                pltpu.make_async_copy(y_hbm.at[sbuf.at[slot, j]],
                                      rows.at[b], gsem.at[b]).start()

            def gather_wait(slot, b):
                pltpu.make_async_copy(y_hbm.at[sbuf.at[slot, 0]],
                                      rows.at[b], gsem.at[b]).wait()

            stage_start(0, 0)
            stage_wait(0)

            @pl.loop(0, ngrp)
            def _(g):
                slot = lax.rem(g, 2)

                @pl.when(g + 1 < ngrp)
                def _():
                    stage_start(g + 1, 1 - slot)

                @pl.loop(0, NBUF)
                def _(j):
                    gather_start(slot, j, j)

                @pl.loop(0, AGRP - NBUF)
                def _(j):
                    b = lax.rem(j, NBUF)
                    gather_wait(slot, b)
                    pltpu.sync_copy(rows.at[b], acc.at[dbuf.at[slot, j]],
                                    add=True)
                    gather_start(slot, j + NBUF, b)

                @pl.loop(AGRP - NBUF, AGRP)
                def _(j):
                    b = lax.rem(j, NBUF)
                    gather_wait(slot, b)
                    pltpu.sync_copy(rows.at[b], acc.at[dbuf.at[slot, j]],
                                    add=True)

                @pl.when(g + 1 < ngrp)
                def _():
                    stage_wait(1 - slot)

            plsc.subcore_barrier()
            pltpu.sync_copy(acc.at[sl], out_hbm.at[sl])

        @pl.when(c == 0)
        def _():
            side(y_r, src_r, dst_r, out_r)

        @pl.when(c == 1)
        def _():
            side(y_l, src_l, dst_l, out_l)

    return sc_aggregate


def _tc_y1(h0, deg, W1):
    def body(h_ref, d_ref, w_ref, y_ref):
        dinv = lax.rsqrt(d_ref[:, 0:1] + 1.0)
        y_ref[...] = jnp.dot(h_ref[...], w_ref[...],
                             preferred_element_type=_f32) * dinv

    return pl.pallas_call(
        body,
        grid=(NBLK,),
        in_specs=[
            pl.BlockSpec((BLK, D), lambda i: (i, 0)),
            pl.BlockSpec((BLK, D), lambda i: (i, 0)),
            pl.BlockSpec((D, D), lambda i: (0, 0)),
        ],
        out_specs=pl.BlockSpec((BLK, D), lambda i: (i, 0)),
        out_shape=jax.ShapeDtypeStruct((NP, D), _f32),
    )(h0, deg, W1)


def _tc_y2(acc1, y1, deg, W2, b1b):
    def body(a_ref, y_ref, d_ref, w_ref, b_ref, o_ref):
        dinv = lax.rsqrt(d_ref[:, 0:1] + 1.0)
        h1 = (a_ref[...] + y_ref[...]) * dinv + b_ref[0:1, :]
        h1 = jnp.maximum(h1, 0.0)
        o_ref[...] = jnp.dot(h1, w_ref[...], preferred_element_type=_f32) * dinv

    return pl.pallas_call(
        body,
        grid=(NBLK,),
        in_specs=[
            pl.BlockSpec((BLK, D), lambda i: (i, 0)),
            pl.BlockSpec((BLK, D), lambda i: (i, 0)),
            pl.BlockSpec((BLK, D), lambda i: (i, 0)),
            pl.BlockSpec((D, D), lambda i: (0, 0)),
            pl.BlockSpec((8, D), lambda i: (0, 0)),
        ],
        out_specs=pl.BlockSpec((BLK, D), lambda i: (i, 0)),
        out_shape=jax.ShapeDtypeStruct((NP, D), _f32),
    )(acc1, y1, deg, W2, b1b)


def _tc_epilogue(acc_r, y_r, deg_r, batch_r, acc_l, y_l, deg_l, batch_l,
                 b2b, Wfc8, bfc8):
    def body(ar, yr, dr, br, al, yl, dl, bl, b2_ref, w_ref, bf_ref, o_ref,
             s_r, c_r, s_l, c_l):
        i = pl.program_id(0)

        @pl.when(i == 0)
        def _():
            s_r[...] = jnp.zeros_like(s_r)
            c_r[...] = jnp.zeros_like(c_r)
            s_l[...] = jnp.zeros_like(s_l)
            c_l[...] = jnp.zeros_like(c_l)

        def side(a_ref, y_ref, d_ref, b_ref, s_scr, c_scr):
            dinv = lax.rsqrt(d_ref[:, 0:1] + 1.0)
            h2 = (a_ref[...] + y_ref[...]) * dinv + b2_ref[0:1, :]
            bid = b_ref[0]
            oh = (lax.broadcasted_iota(_i32, (B, BLK), 0) == bid).astype(_f32)
            s_scr[...] += jnp.dot(oh, h2, preferred_element_type=_f32)
            c_scr[...] += jnp.sum(oh, axis=1, keepdims=True)

        side(ar, yr, dr, br, s_r, c_r)
        side(al, yl, dl, bl, s_l, c_l)

        @pl.when(i == NBLK - 1)
        def _():
            mr = s_r[...] / jnp.maximum(c_r[...], 1.0)
            ml = s_l[...] / jnp.maximum(c_l[...], 1.0)
            hcat = jnp.concatenate([mr, ml], axis=1)
            out = lax.dot_general(hcat, w_ref[...],
                                  (((1,), (1,)), ((), ())),
                                  preferred_element_type=_f32)
            o_ref[...] = out + bf_ref[0:1, :]

    node_spec = pl.BlockSpec((BLK, D), lambda i: (i, 0))
    batch_spec = pl.BlockSpec((1, 1, BLK), lambda i: (i, 0, 0))
    return pl.pallas_call(
        body,
        grid=(NBLK,),
        in_specs=[
            node_spec, node_spec, node_spec, batch_spec,
            node_spec, node_spec, node_spec, batch_spec,
            pl.BlockSpec((8, D), lambda i: (0, 0)),
            pl.BlockSpec((8, 2 * D), lambda i: (0, 0)),
            pl.BlockSpec((8, 8), lambda i: (0, 0)),
        ],
        out_specs=pl.BlockSpec((B, 8), lambda i: (0, 0)),
        out_shape=jax.ShapeDtypeStruct((B, 8), _f32),
        scratch_shapes=[
            pltpu.VMEM((B, D), _f32),
            pltpu.VMEM((B, 1), _f32),
            pltpu.VMEM((B, D), _f32),
            pltpu.VMEM((B, 1), _f32),
        ],
    )(acc_r, y_r, deg_r, batch_r, acc_l, y_l, deg_l, batch_l, b2b, Wfc8, bfc8)


def _prep_nodes(x):
    xp = jnp.concatenate([x.astype(_i32), jnp.zeros((NP - N,), _i32)])
    return xp.reshape(NS, NCH_N, CH)


def _prep_edges(ei):
    src = jnp.concatenate([ei[0].astype(_i32), jnp.zeros((EP - E,), _i32)])
    dst = jnp.concatenate([ei[1].astype(_i32), jnp.full((EP - E,), N, _i32)])
    return src.reshape(EP // CH, CH), dst.reshape(EP // CH, CH)


def _prep_batch(b):
    bp = jnp.concatenate([b.astype(_i32), jnp.full((NP - N,), B + 7, _i32)])
    return bp.reshape(NBLK, 1, BLK)


def kernel(receptor_x, receptor_edge_index, receptor_batch,
           ligand_x, ligand_edge_index, ligand_batch,
           emb_table, W1, b1, W2, b2, Wfc, bfc):
    x_r = _prep_nodes(receptor_x)
    x_l = _prep_nodes(ligand_x)
    src_r, dst_r = _prep_edges(receptor_edge_index)
    src_l, dst_l = _prep_edges(ligand_edge_index)
    batch_r = _prep_batch(receptor_batch)
    batch_l = _prep_batch(ligand_batch)

    zeros = jnp.zeros((NP, D), _f32)
    ones = jnp.ones((CH, D), _f32)
    b1b = jnp.tile(b1[None, :], (8, 1))
    b2b = jnp.tile(b2[None, :], (8, 1))
    Wfc8 = jnp.concatenate([Wfc, jnp.zeros((2, 2 * D), _f32)], axis=0)
    bfc8 = jnp.tile(jnp.concatenate([bfc, jnp.zeros((2,), _f32)])[None, :],
                    (8, 1))

    h0_r, h0_l, deg_r, deg_l = _build_sc_emb_deg()(
        x_r, x_l, dst_r, dst_l, emb_table, zeros, ones)

    y1_r = _tc_y1(h0_r, deg_r, W1)
    y1_l = _tc_y1(h0_l, deg_l, W1)

    acc1_r, acc1_l = _build_sc_aggregate()(
        y1_r, y1_l, src_r, src_l, dst_r, dst_l, zeros)

    y2_r = _tc_y2(acc1_r, y1_r, deg_r, W2, b1b)
    y2_l = _tc_y2(acc1_l, y1_l, deg_l, W2, b1b)

    acc2_r, acc2_l = _build_sc_aggregate()(
        y2_r, y2_l, src_r, src_l, dst_r, dst_l, zeros)

    out8 = _tc_epilogue(acc2_r, y2_r, deg_r, batch_r,
                        acc2_l, y2_l, deg_l, batch_l,
                        b2b, Wfc8, bfc8)
    return (out8[:, :3], out8[:, 3:6])


# speedup vs baseline: 74.9069x; 1.1479x over previous
import functools

import jax
import jax.numpy as jnp
from jax import lax
from jax.experimental import pallas as pl
from jax.experimental.pallas import tpu as pltpu
from jax.experimental.pallas import tpu_sc as plsc

N = 100000
E = 3200000
B = 128
D = 16
VOCAB = 1032

NS = 16
CH = 128
NP = 100352
EP = 3276800
NCH_N = NP // (NS * CH)
NCH_E = EP // (NS * CH)
AGRP = 16
NBUF = 8
GLAG = 4
DLAG = 16
ROWS_PER_TILE = NP // NS
BLK = 1024
NBLK = NP // BLK

_f32 = jnp.float32
_i32 = jnp.int32

_SC_PARAMS = pltpu.CompilerParams(use_tc_tiling_on_sc=False)


@functools.cache
def _build_sc_emb_deg():
    @functools.partial(
        pl.kernel,
        out_type=(
            jax.ShapeDtypeStruct((NP, D), _f32),
            jax.ShapeDtypeStruct((NP, D), _f32),
            jax.ShapeDtypeStruct((NP, D), _f32),
            jax.ShapeDtypeStruct((NP, D), _f32),
        ),
        mesh=plsc.VectorSubcoreMesh(core_axis_name="c", subcore_axis_name="s"),
        compiler_params=_SC_PARAMS,
        scratch_types=[
            pltpu.VMEM_SHARED((NP, D), _f32),
            pltpu.VMEM((NCH_N, CH), _i32),
            pltpu.VMEM((3, AGRP, CH), _i32),
            pltpu.VMEM((2, CH, D), _f32),
            pltpu.VMEM((CH, D), _f32),
            pltpu.SemaphoreType.DMA((2,)),
            pltpu.SemaphoreType.DMA,
            pltpu.SemaphoreType.DMA,
        ],
    )
    def sc_emb_deg(x_r, x_l, dst_r, dst_l, emb, zeros, ones,
                   h0_r, h0_l, deg_r, deg_l,
                   acc, xbuf, ibuf, rows, ones_v, gsem, ssem, dsem):
        c = lax.axis_index("c")
        t = lax.axis_index("s")
        sl = pl.ds(t * ROWS_PER_TILE, ROWS_PER_TILE)
        ngrp = NCH_E // AGRP

        def side(x_hbm, dst_hbm, h0_hbm, deg_hbm):
            pltpu.sync_copy(zeros.at[sl], acc.at[sl])
            pltpu.sync_copy(ones, ones_v)
            pltpu.sync_copy(x_hbm.at[t], xbuf)

            def h0_start(k, b):
                pltpu.make_async_copy(emb.at[xbuf.at[k]], rows.at[b],
                                      gsem.at[b]).start()

            def h0_wait(b):
                pltpu.make_async_copy(emb.at[xbuf.at[0]], rows.at[b],
                                      gsem.at[b]).wait()

            h0_start(0, 0)

            @pl.loop(0, NCH_N)
            def _(k):
                b = lax.rem(k, 2)

                @pl.when(k + 1 < NCH_N)
                def _():
                    h0_start(k + 1, 1 - b)

                h0_wait(b)
                pltpu.sync_copy(rows.at[b],
                                h0_hbm.at[pl.ds((t * NCH_N + k) * CH, CH)])

            plsc.subcore_barrier()

            def stage_start(g):
                base = t * NCH_E + g * AGRP
                pltpu.make_async_copy(dst_hbm.at[pl.ds(base, AGRP)],
                                      ibuf.at[lax.rem(g, 3)], ssem).start()

            def stage_wait(g):
                pltpu.make_async_copy(dst_hbm.at[pl.ds(t * NCH_E, AGRP)],
                                      ibuf.at[lax.rem(g, 3)], ssem).wait()

            def dscat_start(g, r):
                pltpu.make_async_copy(
                    ones_v, acc.at[ibuf.at[lax.rem(g, 3), r]], dsem,
                ).start(add=True)

            def dscat_wait():
                pltpu.make_async_copy(ones_v, acc.at[ibuf.at[0, 0]],
                                      dsem).wait()

            stage_start(0)
            stage_wait(0)
            stage_start(1)

            @pl.loop(0, NCH_E)
            def _(i):
                g = lax.div(i, AGRP)
                r = lax.rem(i, AGRP)

                @pl.when((r == 0) & (g > 0))
                def _():
                    stage_wait(g)

                    @pl.when(g + 1 < ngrp)
                    def _():
                        stage_start(g + 1)

                dscat_start(g, r)

                @pl.when(i >= DLAG)
                def _():
                    dscat_wait()

            @pl.loop(0, DLAG)
            def _(i):
                dscat_wait()

            plsc.subcore_barrier()
            pltpu.sync_copy(acc.at[sl], deg_hbm.at[sl])

        @pl.when(c == 0)
        def _():
            side(x_r, dst_r, h0_r, deg_r)

        @pl.when(c == 1)
        def _():
            side(x_l, dst_l, h0_l, deg_l)

    return sc_emb_deg


@functools.cache
def _build_sc_aggregate():
    @functools.partial(
        pl.kernel,
        out_type=(
            jax.ShapeDtypeStruct((NP, D), _f32),
            jax.ShapeDtypeStruct((NP, D), _f32),
        ),
        mesh=plsc.VectorSubcoreMesh(core_axis_name="c", subcore_axis_name="s"),
        compiler_params=_SC_PARAMS,
        scratch_types=[
            pltpu.VMEM_SHARED((NP, D), _f32),
            pltpu.VMEM((3, AGRP, CH), _i32),
            pltpu.VMEM((3, AGRP, CH), _i32),
            pltpu.VMEM((NBUF, CH, D), _f32),
            pltpu.SemaphoreType.DMA((NBUF,)),
            pltpu.SemaphoreType.DMA((NBUF,)),
            pltpu.SemaphoreType.DMA,
        ],
    )
    def sc_aggregate(y_r, y_l, src_r, src_l, dst_r, dst_l, zeros,
                     out_r, out_l,
                     acc, sbuf, dbuf, rows, gsem, ksem, ssem):
        c = lax.axis_index("c")
        t = lax.axis_index("s")
        sl = pl.ds(t * ROWS_PER_TILE, ROWS_PER_TILE)
        ngrp = NCH_E // AGRP
        m = NCH_E

        def side(y_hbm, src_hbm, dst_hbm, out_hbm):
            pltpu.sync_copy(zeros.at[sl], acc.at[sl])
            plsc.subcore_barrier()

            def stage_start(g):
                base = t * NCH_E + g * AGRP
                slot = lax.rem(g, 3)
                pltpu.make_async_copy(src_hbm.at[pl.ds(base, AGRP)],
                                      sbuf.at[slot], ssem).start()
                pltpu.make_async_copy(dst_hbm.at[pl.ds(base, AGRP)],
                                      dbuf.at[slot], ssem).start()

            def stage_wait(g):
                slot = lax.rem(g, 3)
                pltpu.make_async_copy(src_hbm.at[pl.ds(t * NCH_E, AGRP)],
                                      sbuf.at[slot], ssem).wait()
                pltpu.make_async_copy(dst_hbm.at[pl.ds(t * NCH_E, AGRP)],
                                      dbuf.at[slot], ssem).wait()

            def gather_start(i, b):
                pltpu.make_async_copy(
                    y_hbm.at[sbuf.at[lax.rem(lax.div(i, AGRP), 3),
                                     lax.rem(i, AGRP)]],
                    rows.at[b], gsem.at[b]).start()

            def gather_wait(b):
                pltpu.make_async_copy(y_hbm.at[sbuf.at[0, 0]],
                                      rows.at[b], gsem.at[b]).wait()

            def scat_start(i, b):
                pltpu.make_async_copy(
                    rows.at[b],
                    acc.at[dbuf.at[lax.rem(lax.div(i, AGRP), 3),
                                   lax.rem(i, AGRP)]],
                    ksem.at[b]).start(add=True)

            def scat_wait(b):
                pltpu.make_async_copy(rows.at[b], acc.at[dbuf.at[0, 0]],
                                      ksem.at[b]).wait()

            stage_start(0)
            stage_wait(0)
            stage_start(1)

            @pl.loop(0, m)
            def _(i):
                g = lax.div(i, AGRP)
                r = lax.rem(i, AGRP)

                @pl.when((r == 0) & (g > 0))
                def _():
                    stage_wait(g)

                    @pl.when(g + 1 < ngrp)
                    def _():
                        stage_start(g + 1)

                b = lax.rem(i, NBUF)

                @pl.when(i >= NBUF)
                def _():
                    scat_wait(b)

                gather_start(i, b)

                @pl.when(i >= GLAG)
                def _():
                    b2 = lax.rem(i - GLAG, NBUF)
                    gather_wait(b2)
                    scat_start(i - GLAG, b2)

            @pl.loop(m, m + GLAG)
            def _(i):
                b2 = lax.rem(i - GLAG, NBUF)
                gather_wait(b2)
                scat_start(i - GLAG, b2)

            @pl.loop(0, NBUF)
            def _(b):
                scat_wait(b)

            plsc.subcore_barrier()
            pltpu.sync_copy(acc.at[sl], out_hbm.at[sl])

        @pl.when(c == 0)
        def _():
            side(y_r, src_r, dst_r, out_r)

        @pl.when(c == 1)
        def _():
            side(y_l, src_l, dst_l, out_l)

    return sc_aggregate


def _tc_y1(h0, deg, W1):
    def body(h_ref, d_ref, w_ref, y_ref):
        dinv = lax.rsqrt(d_ref[:, 0:1] + 1.0)
        y_ref[...] = jnp.dot(h_ref[...], w_ref[...],
                             preferred_element_type=_f32) * dinv

    return pl.pallas_call(
        body,
        grid=(NBLK,),
        in_specs=[
            pl.BlockSpec((BLK, D), lambda i: (i, 0)),
            pl.BlockSpec((BLK, D), lambda i: (i, 0)),
            pl.BlockSpec((D, D), lambda i: (0, 0)),
        ],
        out_specs=pl.BlockSpec((BLK, D), lambda i: (i, 0)),
        out_shape=jax.ShapeDtypeStruct((NP, D), _f32),
    )(h0, deg, W1)


def _tc_y2(acc1, y1, deg, W2, b1b):
    def body(a_ref, y_ref, d_ref, w_ref, b_ref, o_ref):
        dinv = lax.rsqrt(d_ref[:, 0:1] + 1.0)
        h1 = (a_ref[...] + y_ref[...]) * dinv + b_ref[0:1, :]
        h1 = jnp.maximum(h1, 0.0)
        o_ref[...] = jnp.dot(h1, w_ref[...], preferred_element_type=_f32) * dinv

    return pl.pallas_call(
        body,
        grid=(NBLK,),
        in_specs=[
            pl.BlockSpec((BLK, D), lambda i: (i, 0)),
            pl.BlockSpec((BLK, D), lambda i: (i, 0)),
            pl.BlockSpec((BLK, D), lambda i: (i, 0)),
            pl.BlockSpec((D, D), lambda i: (0, 0)),
            pl.BlockSpec((8, D), lambda i: (0, 0)),
        ],
        out_specs=pl.BlockSpec((BLK, D), lambda i: (i, 0)),
        out_shape=jax.ShapeDtypeStruct((NP, D), _f32),
    )(acc1, y1, deg, W2, b1b)


def _tc_epilogue(acc_r, y_r, deg_r, batch_r, acc_l, y_l, deg_l, batch_l,
                 b2b, Wfc8, bfc8):
    def body(ar, yr, dr, br, al, yl, dl, bl, b2_ref, w_ref, bf_ref, o_ref,
             s_r, c_r, s_l, c_l):
        i = pl.program_id(0)

        @pl.when(i == 0)
        def _():
            s_r[...] = jnp.zeros_like(s_r)
            c_r[...] = jnp.zeros_like(c_r)
            s_l[...] = jnp.zeros_like(s_l)
            c_l[...] = jnp.zeros_like(c_l)

        def side(a_ref, y_ref, d_ref, b_ref, s_scr, c_scr):
            dinv = lax.rsqrt(d_ref[:, 0:1] + 1.0)
            h2 = (a_ref[...] + y_ref[...]) * dinv + b2_ref[0:1, :]
            bid = b_ref[0]
            oh = (lax.broadcasted_iota(_i32, (B, BLK), 0) == bid).astype(_f32)
            s_scr[...] += jnp.dot(oh, h2, preferred_element_type=_f32)
            c_scr[...] += jnp.sum(oh, axis=1, keepdims=True)

        side(ar, yr, dr, br, s_r, c_r)
        side(al, yl, dl, bl, s_l, c_l)

        @pl.when(i == NBLK - 1)
        def _():
            mr = s_r[...] / jnp.maximum(c_r[...], 1.0)
            ml = s_l[...] / jnp.maximum(c_l[...], 1.0)
            hcat = jnp.concatenate([mr, ml], axis=1)
            out = lax.dot_general(hcat, w_ref[...],
                                  (((1,), (1,)), ((), ())),
                                  preferred_element_type=_f32)
            o_ref[...] = out + bf_ref[0:1, :]

    node_spec = pl.BlockSpec((BLK, D), lambda i: (i, 0))
    batch_spec = pl.BlockSpec((1, 1, BLK), lambda i: (i, 0, 0))
    return pl.pallas_call(
        body,
        grid=(NBLK,),
        in_specs=[
            node_spec, node_spec, node_spec, batch_spec,
            node_spec, node_spec, node_spec, batch_spec,
            pl.BlockSpec((8, D), lambda i: (0, 0)),
            pl.BlockSpec((8, 2 * D), lambda i: (0, 0)),
            pl.BlockSpec((8, 8), lambda i: (0, 0)),
        ],
        out_specs=pl.BlockSpec((B, 8), lambda i: (0, 0)),
        out_shape=jax.ShapeDtypeStruct((B, 8), _f32),
        scratch_shapes=[
            pltpu.VMEM((B, D), _f32),
            pltpu.VMEM((B, 1), _f32),
            pltpu.VMEM((B, D), _f32),
            pltpu.VMEM((B, 1), _f32),
        ],
    )(acc_r, y_r, deg_r, batch_r, acc_l, y_l, deg_l, batch_l, b2b, Wfc8, bfc8)


def _prep_nodes(x):
    xp = jnp.concatenate([x.astype(_i32), jnp.zeros((NP - N,), _i32)])
    return xp.reshape(NS, NCH_N, CH)


def _prep_edges(ei):
    src = jnp.concatenate([ei[0].astype(_i32), jnp.zeros((EP - E,), _i32)])
    dst = jnp.concatenate([ei[1].astype(_i32), jnp.full((EP - E,), N, _i32)])
    return src.reshape(EP // CH, CH), dst.reshape(EP // CH, CH)


def _prep_batch(b):
    bp = jnp.concatenate([b.astype(_i32), jnp.full((NP - N,), B + 7, _i32)])
    return bp.reshape(NBLK, 1, BLK)


def kernel(receptor_x, receptor_edge_index, receptor_batch,
           ligand_x, ligand_edge_index, ligand_batch,
           emb_table, W1, b1, W2, b2, Wfc, bfc):
    x_r = _prep_nodes(receptor_x)
    x_l = _prep_nodes(ligand_x)
    src_r, dst_r = _prep_edges(receptor_edge_index)
    src_l, dst_l = _prep_edges(ligand_edge_index)
    batch_r = _prep_batch(receptor_batch)
    batch_l = _prep_batch(ligand_batch)

    zeros = jnp.zeros((NP, D), _f32)
    ones = jnp.ones((CH, D), _f32)
    b1b = jnp.tile(b1[None, :], (8, 1))
    b2b = jnp.tile(b2[None, :], (8, 1))
    Wfc8 = jnp.concatenate([Wfc, jnp.zeros((2, 2 * D), _f32)], axis=0)
    bfc8 = jnp.tile(jnp.concatenate([bfc, jnp.zeros((2,), _f32)])[None, :],
                    (8, 1))

    h0_r, h0_l, deg_r, deg_l = _build_sc_emb_deg()(
        x_r, x_l, dst_r, dst_l, emb_table, zeros, ones)

    y1_r = _tc_y1(h0_r, deg_r, W1)
    y1_l = _tc_y1(h0_l, deg_l, W1)

    acc1_r, acc1_l = _build_sc_aggregate()(
        y1_r, y1_l, src_r, src_l, dst_r, dst_l, zeros)

    y2_r = _tc_y2(acc1_r, y1_r, deg_r, W2, b1b)
    y2_l = _tc_y2(acc1_l, y1_l, deg_l, W2, b1b)

    acc2_r, acc2_l = _build_sc_aggregate()(
        y2_r, y2_l, src_r, src_l, dst_r, dst_l, zeros)

    out8 = _tc_epilogue(acc2_r, y2_r, deg_r, batch_r,
                        acc2_l, y2_l, deg_l, batch_l,
                        b2b, Wfc8, bfc8)
    return (out8[:, :3], out8[:, 3:6])


# speedup vs baseline: 75.2473x; 1.0045x over previous
import functools

import jax
import jax.numpy as jnp
from jax import lax
from jax.experimental import pallas as pl
from jax.experimental.pallas import tpu as pltpu
from jax.experimental.pallas import tpu_sc as plsc

N = 100000
E = 3200000
B = 128
D = 16
VOCAB = 1032

NS = 16
CH = 128
CHE = 256
NP = 100352
EP = 3276800
NCH_N = NP // (NS * CH)
NCH_E = EP // (NS * CHE)
AGRP = 8
NBUF = 4
GLAG = 2
DLAG = 8
ROWS_PER_TILE = NP // NS
BLK = 1024
NBLK = NP // BLK

_f32 = jnp.float32
_i32 = jnp.int32

_SC_PARAMS = pltpu.CompilerParams(use_tc_tiling_on_sc=False)


@functools.cache
def _build_sc_emb_deg():
    @functools.partial(
        pl.kernel,
        out_type=(
            jax.ShapeDtypeStruct((NP, D), _f32),
            jax.ShapeDtypeStruct((NP, D), _f32),
            jax.ShapeDtypeStruct((NP, D), _f32),
            jax.ShapeDtypeStruct((NP, D), _f32),
        ),
        mesh=plsc.VectorSubcoreMesh(core_axis_name="c", subcore_axis_name="s"),
        compiler_params=_SC_PARAMS,
        scratch_types=[
            pltpu.VMEM_SHARED((NP, D), _f32),
            pltpu.VMEM((NCH_N, CH), _i32),
            pltpu.VMEM((3, AGRP, CHE), _i32),
            pltpu.VMEM((2, CH, D), _f32),
            pltpu.VMEM((CHE, D), _f32),
            pltpu.SemaphoreType.DMA((2,)),
            pltpu.SemaphoreType.DMA,
            pltpu.SemaphoreType.DMA,
        ],
    )
    def sc_emb_deg(x_r, x_l, dst_r, dst_l, emb, zeros, ones,
                   h0_r, h0_l, deg_r, deg_l,
                   acc, xbuf, ibuf, rows, ones_v, gsem, ssem, dsem):
        c = lax.axis_index("c")
        t = lax.axis_index("s")
        sl = pl.ds(t * ROWS_PER_TILE, ROWS_PER_TILE)
        ngrp = NCH_E // AGRP

        def side(x_hbm, dst_hbm, h0_hbm, deg_hbm):
            pltpu.sync_copy(zeros.at[sl], acc.at[sl])
            pltpu.sync_copy(ones, ones_v)
            pltpu.sync_copy(x_hbm.at[t], xbuf)

            def h0_start(k, b):
                pltpu.make_async_copy(emb.at[xbuf.at[k]], rows.at[b],
                                      gsem.at[b]).start()

            def h0_wait(b):
                pltpu.make_async_copy(emb.at[xbuf.at[0]], rows.at[b],
                                      gsem.at[b]).wait()

            h0_start(0, 0)

            @pl.loop(0, NCH_N)
            def _(k):
                b = lax.rem(k, 2)

                @pl.when(k + 1 < NCH_N)
                def _():
                    h0_start(k + 1, 1 - b)

                h0_wait(b)
                pltpu.sync_copy(rows.at[b],
                                h0_hbm.at[pl.ds((t * NCH_N + k) * CH, CH)])

            plsc.subcore_barrier()

            def stage_start(g):
                base = t * NCH_E + g * AGRP
                pltpu.make_async_copy(dst_hbm.at[pl.ds(base, AGRP)],
                                      ibuf.at[lax.rem(g, 3)], ssem).start()

            def stage_wait(g):
                pltpu.make_async_copy(dst_hbm.at[pl.ds(t * NCH_E, AGRP)],
                                      ibuf.at[lax.rem(g, 3)], ssem).wait()

            def dscat_start(g, r):
                pltpu.make_async_copy(
                    ones_v, acc.at[ibuf.at[lax.rem(g, 3), r]], dsem,
                ).start(add=True)

            def dscat_wait():
                pltpu.make_async_copy(ones_v, acc.at[ibuf.at[0, 0]],
                                      dsem).wait()

            stage_start(0)
            stage_wait(0)
            stage_start(1)

            @pl.loop(0, NCH_E)
            def _(i):
                g = lax.div(i, AGRP)
                r = lax.rem(i, AGRP)

                @pl.when((r == 0) & (g > 0))
                def _():
                    stage_wait(g)

                    @pl.when(g + 1 < ngrp)
                    def _():
                        stage_start(g + 1)

                dscat_start(g, r)

                @pl.when(i >= DLAG)
                def _():
                    dscat_wait()

            @pl.loop(0, DLAG)
            def _(i):
                dscat_wait()

            plsc.subcore_barrier()
            pltpu.sync_copy(acc.at[sl], deg_hbm.at[sl])

        @pl.when(c == 0)
        def _():
            side(x_r, dst_r, h0_r, deg_r)

        @pl.when(c == 1)
        def _():
            side(x_l, dst_l, h0_l, deg_l)

    return sc_emb_deg


@functools.cache
def _build_sc_aggregate():
    @functools.partial(
        pl.kernel,
        out_type=(
            jax.ShapeDtypeStruct((NP, D), _f32),
            jax.ShapeDtypeStruct((NP, D), _f32),
        ),
        mesh=plsc.VectorSubcoreMesh(core_axis_name="c", subcore_axis_name="s"),
        compiler_params=_SC_PARAMS,
        scratch_types=[
            pltpu.VMEM_SHARED((NP, D), _f32),
            pltpu.VMEM((3, AGRP, CHE), _i32),
            pltpu.VMEM((3, AGRP, CHE), _i32),
            pltpu.VMEM((NBUF, CHE, D), _f32),
            pltpu.SemaphoreType.DMA((NBUF,)),
            pltpu.SemaphoreType.DMA((NBUF,)),
            pltpu.SemaphoreType.DMA,
        ],
    )
    def sc_aggregate(y_r, y_l, src_r, src_l, dst_r, dst_l, zeros,
                     out_r, out_l,
                     acc, sbuf, dbuf, rows, gsem, ksem, ssem):
        c = lax.axis_index("c")
        t = lax.axis_index("s")
        sl = pl.ds(t * ROWS_PER_TILE, ROWS_PER_TILE)
        ngrp = NCH_E // AGRP
        m = NCH_E

        def side(y_hbm, src_hbm, dst_hbm, out_hbm):
            pltpu.sync_copy(zeros.at[sl], acc.at[sl])
            plsc.subcore_barrier()

            def stage_start(g):
                base = t * NCH_E + g * AGRP
                slot = lax.rem(g, 3)
                pltpu.make_async_copy(src_hbm.at[pl.ds(base, AGRP)],
                                      sbuf.at[slot], ssem).start()
                pltpu.make_async_copy(dst_hbm.at[pl.ds(base, AGRP)],
                                      dbuf.at[slot], ssem).start()

            def stage_wait(g):
                slot = lax.rem(g, 3)
                pltpu.make_async_copy(src_hbm.at[pl.ds(t * NCH_E, AGRP)],
                                      sbuf.at[slot], ssem).wait()
                pltpu.make_async_copy(dst_hbm.at[pl.ds(t * NCH_E, AGRP)],
                                      dbuf.at[slot], ssem).wait()

            def gather_start(i, b):
                pltpu.make_async_copy(
                    y_hbm.at[sbuf.at[lax.rem(lax.div(i, AGRP), 3),
                                     lax.rem(i, AGRP)]],
                    rows.at[b], gsem.at[b]).start()

            def gather_wait(b):
                pltpu.make_async_copy(y_hbm.at[sbuf.at[0, 0]],
                                      rows.at[b], gsem.at[b]).wait()

            def scat_start(i, b):
                pltpu.make_async_copy(
                    rows.at[b],
                    acc.at[dbuf.at[lax.rem(lax.div(i, AGRP), 3),
                                   lax.rem(i, AGRP)]],
                    ksem.at[b]).start(add=True)

            def scat_wait(b):
                pltpu.make_async_copy(rows.at[b], acc.at[dbuf.at[0, 0]],
                                      ksem.at[b]).wait()

            stage_start(0)
            stage_wait(0)
            stage_start(1)

            @pl.loop(0, m)
            def _(i):
                g = lax.div(i, AGRP)
                r = lax.rem(i, AGRP)

                @pl.when((r == 0) & (g > 0))
                def _():
                    stage_wait(g)

                    @pl.when(g + 1 < ngrp)
                    def _():
                        stage_start(g + 1)

                b = lax.rem(i, NBUF)

                @pl.when(i >= NBUF)
                def _():
                    scat_wait(b)

                gather_start(i, b)

                @pl.when(i >= GLAG)
                def _():
                    b2 = lax.rem(i - GLAG, NBUF)
                    gather_wait(b2)
                    scat_start(i - GLAG, b2)

            @pl.loop(m, m + GLAG)
            def _(i):
                b2 = lax.rem(i - GLAG, NBUF)
                gather_wait(b2)
                scat_start(i - GLAG, b2)

            @pl.loop(0, NBUF)
            def _(b):
                scat_wait(b)

            plsc.subcore_barrier()
            pltpu.sync_copy(acc.at[sl], out_hbm.at[sl])

        @pl.when(c == 0)
        def _():
            side(y_r, src_r, dst_r, out_r)

        @pl.when(c == 1)
        def _():
            side(y_l, src_l, dst_l, out_l)

    return sc_aggregate


def _tc_y1(h0, deg, W1):
    def body(h_ref, d_ref, w_ref, y_ref):
        dinv = lax.rsqrt(d_ref[:, 0:1] + 1.0)
        y_ref[...] = jnp.dot(h_ref[...], w_ref[...],
                             preferred_element_type=_f32) * dinv

    return pl.pallas_call(
        body,
        grid=(NBLK,),
        in_specs=[
            pl.BlockSpec((BLK, D), lambda i: (i, 0)),
            pl.BlockSpec((BLK, D), lambda i: (i, 0)),
            pl.BlockSpec((D, D), lambda i: (0, 0)),
        ],
        out_specs=pl.BlockSpec((BLK, D), lambda i: (i, 0)),
        out_shape=jax.ShapeDtypeStruct((NP, D), _f32),
    )(h0, deg, W1)


def _tc_y2(acc1, y1, deg, W2, b1b):
    def body(a_ref, y_ref, d_ref, w_ref, b_ref, o_ref):
        dinv = lax.rsqrt(d_ref[:, 0:1] + 1.0)
        h1 = (a_ref[...] + y_ref[...]) * dinv + b_ref[0:1, :]
        h1 = jnp.maximum(h1, 0.0)
        o_ref[...] = jnp.dot(h1, w_ref[...], preferred_element_type=_f32) * dinv

    return pl.pallas_call(
        body,
        grid=(NBLK,),
        in_specs=[
            pl.BlockSpec((BLK, D), lambda i: (i, 0)),
            pl.BlockSpec((BLK, D), lambda i: (i, 0)),
            pl.BlockSpec((BLK, D), lambda i: (i, 0)),
            pl.BlockSpec((D, D), lambda i: (0, 0)),
            pl.BlockSpec((8, D), lambda i: (0, 0)),
        ],
        out_specs=pl.BlockSpec((BLK, D), lambda i: (i, 0)),
        out_shape=jax.ShapeDtypeStruct((NP, D), _f32),
    )(acc1, y1, deg, W2, b1b)


def _tc_epilogue(acc_r, y_r, deg_r, batch_r, acc_l, y_l, deg_l, batch_l,
                 b2b, Wfc8, bfc8):
    def body(ar, yr, dr, br, al, yl, dl, bl, b2_ref, w_ref, bf_ref, o_ref,
             s_r, c_r, s_l, c_l):
        i = pl.program_id(0)

        @pl.when(i == 0)
        def _():
            s_r[...] = jnp.zeros_like(s_r)
            c_r[...] = jnp.zeros_like(c_r)
            s_l[...] = jnp.zeros_like(s_l)
            c_l[...] = jnp.zeros_like(c_l)

        def side(a_ref, y_ref, d_ref, b_ref, s_scr, c_scr):
            dinv = lax.rsqrt(d_ref[:, 0:1] + 1.0)
            h2 = (a_ref[...] + y_ref[...]) * dinv + b2_ref[0:1, :]
            bid = b_ref[0]
            oh = (lax.broadcasted_iota(_i32, (B, BLK), 0) == bid).astype(_f32)
            s_scr[...] += jnp.dot(oh, h2, preferred_element_type=_f32)
            c_scr[...] += jnp.sum(oh, axis=1, keepdims=True)

        side(ar, yr, dr, br, s_r, c_r)
        side(al, yl, dl, bl, s_l, c_l)

        @pl.when(i == NBLK - 1)
        def _():
            mr = s_r[...] / jnp.maximum(c_r[...], 1.0)
            ml = s_l[...] / jnp.maximum(c_l[...], 1.0)
            hcat = jnp.concatenate([mr, ml], axis=1)
            out = lax.dot_general(hcat, w_ref[...],
                                  (((1,), (1,)), ((), ())),
                                  preferred_element_type=_f32)
            o_ref[...] = out + bf_ref[0:1, :]

    node_spec = pl.BlockSpec((BLK, D), lambda i: (i, 0))
    batch_spec = pl.BlockSpec((1, 1, BLK), lambda i: (i, 0, 0))
    return pl.pallas_call(
        body,
        grid=(NBLK,),
        in_specs=[
            node_spec, node_spec, node_spec, batch_spec,
            node_spec, node_spec, node_spec, batch_spec,
            pl.BlockSpec((8, D), lambda i: (0, 0)),
            pl.BlockSpec((8, 2 * D), lambda i: (0, 0)),
            pl.BlockSpec((8, 8), lambda i: (0, 0)),
        ],
        out_specs=pl.BlockSpec((B, 8), lambda i: (0, 0)),
        out_shape=jax.ShapeDtypeStruct((B, 8), _f32),
        scratch_shapes=[
            pltpu.VMEM((B, D), _f32),
            pltpu.VMEM((B, 1), _f32),
            pltpu.VMEM((B, D), _f32),
            pltpu.VMEM((B, 1), _f32),
        ],
    )(acc_r, y_r, deg_r, batch_r, acc_l, y_l, deg_l, batch_l, b2b, Wfc8, bfc8)


def _prep_nodes(x):
    xp = jnp.concatenate([x.astype(_i32), jnp.zeros((NP - N,), _i32)])
    return xp.reshape(NS, NCH_N, CH)


def _prep_edges(ei):
    src = jnp.concatenate([ei[0].astype(_i32), jnp.zeros((EP - E,), _i32)])
    dst = jnp.concatenate([ei[1].astype(_i32), jnp.full((EP - E,), N, _i32)])
    return src.reshape(EP // CHE, CHE), dst.reshape(EP // CHE, CHE)


def _prep_batch(b):
    bp = jnp.concatenate([b.astype(_i32), jnp.full((NP - N,), B + 7, _i32)])
    return bp.reshape(NBLK, 1, BLK)


def kernel(receptor_x, receptor_edge_index, receptor_batch,
           ligand_x, ligand_edge_index, ligand_batch,
           emb_table, W1, b1, W2, b2, Wfc, bfc):
    x_r = _prep_nodes(receptor_x)
    x_l = _prep_nodes(ligand_x)
    src_r, dst_r = _prep_edges(receptor_edge_index)
    src_l, dst_l = _prep_edges(ligand_edge_index)
    batch_r = _prep_batch(receptor_batch)
    batch_l = _prep_batch(ligand_batch)

    zeros = jnp.zeros((NP, D), _f32)
    ones = jnp.ones((CHE, D), _f32)
    b1b = jnp.tile(b1[None, :], (8, 1))
    b2b = jnp.tile(b2[None, :], (8, 1))
    Wfc8 = jnp.concatenate([Wfc, jnp.zeros((2, 2 * D), _f32)], axis=0)
    bfc8 = jnp.tile(jnp.concatenate([bfc, jnp.zeros((2,), _f32)])[None, :],
                    (8, 1))

    h0_r, h0_l, deg_r, deg_l = _build_sc_emb_deg()(
        x_r, x_l, dst_r, dst_l, emb_table, zeros, ones)

    y1_r = _tc_y1(h0_r, deg_r, W1)
    y1_l = _tc_y1(h0_l, deg_l, W1)

    acc1_r, acc1_l = _build_sc_aggregate()(
        y1_r, y1_l, src_r, src_l, dst_r, dst_l, zeros)

    y2_r = _tc_y2(acc1_r, y1_r, deg_r, W2, b1b)
    y2_l = _tc_y2(acc1_l, y1_l, deg_l, W2, b1b)

    acc2_r, acc2_l = _build_sc_aggregate()(
        y2_r, y2_l, src_r, src_l, dst_r, dst_l, zeros)

    out8 = _tc_epilogue(acc2_r, y2_r, deg_r, batch_r,
                        acc2_l, y2_l, deg_l, batch_l,
                        b2b, Wfc8, bfc8)
    return (out8[:, :3], out8[:, 3:6])


# speedup vs baseline: 97.8238x; 1.3000x over previous
import functools

import jax
import jax.numpy as jnp
from jax import lax
from jax.experimental import pallas as pl
from jax.experimental.pallas import tpu as pltpu
from jax.experimental.pallas import tpu_sc as plsc

N = 100000
E = 3200000
B = 128
D = 16
VOCAB = 1032

NS = 16
CH = 128
CHE = 256
NP = 100352
EP = 3276800
NPR = NP // 8
ECH = EP // CHE
NCH_N = NP // (NS * CH)
NCH_E = EP // (NS * CHE)
AGRP = 8
NBUF = 4
GLAG = 2
DLAG = 8
ROWS_PER_TILE = NP // NS
PBLK = 784
NBLK = NPR // PBLK

_f32 = jnp.float32
_i32 = jnp.int32

_SC_PARAMS = pltpu.CompilerParams(use_tc_tiling_on_sc=False)


@functools.cache
def _build_sc_emb_deg():
    @functools.partial(
        pl.kernel,
        out_type=(
            jax.ShapeDtypeStruct((NP, D), _f32),
            jax.ShapeDtypeStruct((NP, D), _f32),
            jax.ShapeDtypeStruct((NP, D), _f32),
            jax.ShapeDtypeStruct((NP, D), _f32),
        ),
        mesh=plsc.VectorSubcoreMesh(core_axis_name="c", subcore_axis_name="s"),
        compiler_params=_SC_PARAMS,
        scratch_types=[
            pltpu.VMEM_SHARED((NP, D), _f32),
            pltpu.VMEM((NCH_N, CH), _i32),
            pltpu.VMEM((3, AGRP, CHE), _i32),
            pltpu.VMEM((2, CH, D), _f32),
            pltpu.VMEM((CHE, D), _f32),
            pltpu.SemaphoreType.DMA((2,)),
            pltpu.SemaphoreType.DMA,
            pltpu.SemaphoreType.DMA,
        ],
    )
    def sc_emb_deg(x_r, x_l, e_r, e_l, emb, zeros, ones,
                   h0_r, h0_l, deg_r, deg_l,
                   acc, xbuf, ibuf, rows, ones_v, gsem, ssem, dsem):
        c = lax.axis_index("c")
        t = lax.axis_index("s")
        sl = pl.ds(t * ROWS_PER_TILE, ROWS_PER_TILE)
        ngrp = NCH_E // AGRP

        def side(x_hbm, e_hbm, h0_hbm, deg_hbm):
            pltpu.sync_copy(zeros.at[sl], acc.at[sl])
            pltpu.sync_copy(ones, ones_v)
            pltpu.sync_copy(x_hbm.at[t], xbuf)

            def h0_start(k, b):
                pltpu.make_async_copy(emb.at[xbuf.at[k]], rows.at[b],
                                      gsem.at[b]).start()

            def h0_wait(b):
                pltpu.make_async_copy(emb.at[xbuf.at[0]], rows.at[b],
                                      gsem.at[b]).wait()

            h0_start(0, 0)

            @pl.loop(0, NCH_N)
            def _(k):
                b = lax.rem(k, 2)

                @pl.when(k + 1 < NCH_N)
                def _():
                    h0_start(k + 1, 1 - b)

                h0_wait(b)
                pltpu.sync_copy(rows.at[b],
                                h0_hbm.at[pl.ds((t * NCH_N + k) * CH, CH)])

            plsc.subcore_barrier()

            def stage_start(g):
                base = t * NCH_E + g * AGRP
                pltpu.make_async_copy(e_hbm.at[1, pl.ds(base, AGRP)],
                                      ibuf.at[lax.rem(g, 3)], ssem).start()

            def stage_wait(g):
                pltpu.make_async_copy(e_hbm.at[1, pl.ds(t * NCH_E, AGRP)],
                                      ibuf.at[lax.rem(g, 3)], ssem).wait()

            def dscat_start(g, r):
                pltpu.make_async_copy(
                    ones_v, acc.at[ibuf.at[lax.rem(g, 3), r]], dsem,
                ).start(add=True)

            def dscat_wait():
                pltpu.make_async_copy(ones_v, acc.at[ibuf.at[0, 0]],
                                      dsem).wait()

            stage_start(0)
            stage_wait(0)
            stage_start(1)

            @pl.loop(0, NCH_E)
            def _(i):
                g = lax.div(i, AGRP)
                r = lax.rem(i, AGRP)

                @pl.when((r == 0) & (g > 0))
                def _():
                    stage_wait(g)

                    @pl.when(g + 1 < ngrp)
                    def _():
                        stage_start(g + 1)

                dscat_start(g, r)

                @pl.when(i >= DLAG)
                def _():
                    dscat_wait()

            @pl.loop(0, DLAG)
            def _(i):
                dscat_wait()

            plsc.subcore_barrier()
            pltpu.sync_copy(acc.at[sl], deg_hbm.at[sl])

        @pl.when(c == 0)
        def _():
            side(x_r, e_r, h0_r, deg_r)

        @pl.when(c == 1)
        def _():
            side(x_l, e_l, h0_l, deg_l)

    return sc_emb_deg


@functools.cache
def _build_sc_aggregate():
    @functools.partial(
        pl.kernel,
        out_type=(
            jax.ShapeDtypeStruct((NP, D), _f32),
            jax.ShapeDtypeStruct((NP, D), _f32),
        ),
        mesh=plsc.VectorSubcoreMesh(core_axis_name="c", subcore_axis_name="s"),
        compiler_params=_SC_PARAMS,
        scratch_types=[
            pltpu.VMEM_SHARED((NP, D), _f32),
            pltpu.VMEM((3, AGRP, CHE), _i32),
            pltpu.VMEM((3, AGRP, CHE), _i32),
            pltpu.VMEM((NBUF, CHE, D), _f32),
            pltpu.SemaphoreType.DMA((NBUF,)),
            pltpu.SemaphoreType.DMA((NBUF,)),
            pltpu.SemaphoreType.DMA,
        ],
    )
    def sc_aggregate(y_r, y_l, e_r, e_l, zeros,
                     out_r, out_l,
                     acc, sbuf, dbuf, rows, gsem, ksem, ssem):
        c = lax.axis_index("c")
        t = lax.axis_index("s")
        sl = pl.ds(t * ROWS_PER_TILE, ROWS_PER_TILE)
        ngrp = NCH_E // AGRP
        m = NCH_E

        def side(y_hbm, e_hbm, out_hbm):
            pltpu.sync_copy(zeros.at[sl], acc.at[sl])
            plsc.subcore_barrier()

            def stage_start(g):
                base = t * NCH_E + g * AGRP
                slot = lax.rem(g, 3)
                pltpu.make_async_copy(e_hbm.at[0, pl.ds(base, AGRP)],
                                      sbuf.at[slot], ssem).start()
                pltpu.make_async_copy(e_hbm.at[1, pl.ds(base, AGRP)],
                                      dbuf.at[slot], ssem).start()

            def stage_wait(g):
                slot = lax.rem(g, 3)
                pltpu.make_async_copy(e_hbm.at[0, pl.ds(t * NCH_E, AGRP)],
                                      sbuf.at[slot], ssem).wait()
                pltpu.make_async_copy(e_hbm.at[1, pl.ds(t * NCH_E, AGRP)],
                                      dbuf.at[slot], ssem).wait()

            def gather_start(i, b):
                pltpu.make_async_copy(
                    y_hbm.at[sbuf.at[lax.rem(lax.div(i, AGRP), 3),
                                     lax.rem(i, AGRP)]],
                    rows.at[b], gsem.at[b]).start()

            def gather_wait(b):
                pltpu.make_async_copy(y_hbm.at[sbuf.at[0, 0]],
                                      rows.at[b], gsem.at[b]).wait()

            def scat_start(i, b):
                pltpu.make_async_copy(
                    rows.at[b],
                    acc.at[dbuf.at[lax.rem(lax.div(i, AGRP), 3),
                                   lax.rem(i, AGRP)]],
                    ksem.at[b]).start(add=True)

            def scat_wait(b):
                pltpu.make_async_copy(rows.at[b], acc.at[dbuf.at[0, 0]],
                                      ksem.at[b]).wait()

            stage_start(0)
            stage_wait(0)
            stage_start(1)

            @pl.loop(0, m)
            def _(i):
                g = lax.div(i, AGRP)
                r = lax.rem(i, AGRP)

                @pl.when((r == 0) & (g > 0))
                def _():
                    stage_wait(g)

                    @pl.when(g + 1 < ngrp)
                    def _():
                        stage_start(g + 1)

                b = lax.rem(i, NBUF)

                @pl.when(i >= NBUF)
                def _():
                    scat_wait(b)

                gather_start(i, b)

                @pl.when(i >= GLAG)
                def _():
                    b2 = lax.rem(i - GLAG, NBUF)
                    gather_wait(b2)
                    scat_start(i - GLAG, b2)

            @pl.loop(m, m + GLAG)
            def _(i):
                b2 = lax.rem(i - GLAG, NBUF)
                gather_wait(b2)
                scat_start(i - GLAG, b2)

            @pl.loop(0, NBUF)
            def _(b):
                scat_wait(b)

            plsc.subcore_barrier()
            pltpu.sync_copy(acc.at[sl], out_hbm.at[sl])

        @pl.when(c == 0)
        def _():
            side(y_r, e_r, out_r)

        @pl.when(c == 1)
        def _():
            side(y_l, e_l, out_l)

    return sc_aggregate


def _tc_y1(h0, deg, W1bd):
    def body(h_ref, d_ref, w_ref, y_ref):
        dinv = lax.rsqrt(d_ref[...] + 1.0)
        y_ref[...] = jnp.dot(h_ref[...], w_ref[...],
                             preferred_element_type=_f32) * dinv

    return pl.pallas_call(
        body,
        grid=(NBLK,),
        in_specs=[
            pl.BlockSpec((PBLK, 128), lambda i: (i, 0)),
            pl.BlockSpec((PBLK, 128), lambda i: (i, 0)),
            pl.BlockSpec((128, 128), lambda i: (0, 0)),
        ],
        out_specs=pl.BlockSpec((PBLK, 128), lambda i: (i, 0)),
        out_shape=jax.ShapeDtypeStruct((NPR, 128), _f32),
    )(h0, deg, W1bd)


def _tc_y2(acc1, y1, deg, W2bd, b1p):
    def body(a_ref, y_ref, d_ref, w_ref, b_ref, o_ref):
        dinv = lax.rsqrt(d_ref[...] + 1.0)
        h1 = (a_ref[...] + y_ref[...]) * dinv + b_ref[0:1, :]
        h1 = jnp.maximum(h1, 0.0)
        o_ref[...] = jnp.dot(h1, w_ref[...], preferred_element_type=_f32) * dinv

    return pl.pallas_call(
        body,
        grid=(NBLK,),
        in_specs=[
            pl.BlockSpec((PBLK, 128), lambda i: (i, 0)),
            pl.BlockSpec((PBLK, 128), lambda i: (i, 0)),
            pl.BlockSpec((PBLK, 128), lambda i: (i, 0)),
            pl.BlockSpec((128, 128), lambda i: (0, 0)),
            pl.BlockSpec((8, 128), lambda i: (0, 0)),
        ],
        out_specs=pl.BlockSpec((PBLK, 128), lambda i: (i, 0)),
        out_shape=jax.ShapeDtypeStruct((NPR, 128), _f32),
    )(acc1, y1, deg, W2bd, b1p)


def _tc_epilogue(acc_r, y_r, deg_r, batch_r, acc_l, y_l, deg_l, batch_l,
                 b2p, Wfc8, bfc8):
    nodes = PBLK * 8

    def body(ar, yr, dr, br, al, yl, dl, bl, b2_ref, w_ref, bf_ref, o_ref,
             s_r, c_r, s_l, c_l):
        i = pl.program_id(0)

        @pl.when(i == 0)
        def _():
            s_r[...] = jnp.zeros_like(s_r)
            c_r[...] = jnp.zeros_like(c_r)
            s_l[...] = jnp.zeros_like(s_l)
            c_l[...] = jnp.zeros_like(c_l)

        def side(a_ref, y_ref, d_ref, b_ref, s_scr, c_scr):
            dinv = lax.rsqrt(d_ref[...] + 1.0)
            h2p = (a_ref[...] + y_ref[...]) * dinv + b2_ref[0:1, :]
            bid8 = b_ref[0]
            s = jnp.zeros((B, D), _f32)
            cnt = jnp.zeros((B, 1), _f32)
            for k in range(8):
                row = bid8[k:k + 1, :]
                oh = (lax.broadcasted_iota(_i32, (B, PBLK), 0)
                      == row).astype(_f32)
                s = s + jnp.dot(oh, h2p[:, 16 * k:16 * (k + 1)],
                                preferred_element_type=_f32)
                cnt = cnt + jnp.sum(oh, axis=1, keepdims=True)
            s_scr[...] += s
            c_scr[...] += cnt

        side(ar, yr, dr, br, s_r, c_r)
        side(al, yl, dl, bl, s_l, c_l)

        @pl.when(i == NBLK - 1)
        def _():
            mr = s_r[...] / jnp.maximum(c_r[...], 1.0)
            ml = s_l[...] / jnp.maximum(c_l[...], 1.0)
            hcat = jnp.concatenate([mr, ml], axis=1)
            out = lax.dot_general(hcat, w_ref[...],
                                  (((1,), (1,)), ((), ())),
                                  preferred_element_type=_f32)
            o_ref[...] = out + bf_ref[0:1, :]

    node_spec = pl.BlockSpec((PBLK, 128), lambda i: (i, 0))
    batch_spec = pl.BlockSpec((1, 8, PBLK), lambda i: (i, 0, 0))
    return pl.pallas_call(
        body,
        grid=(NBLK,),
        in_specs=[
            node_spec, node_spec, node_spec, batch_spec,
            node_spec, node_spec, node_spec, batch_spec,
            pl.BlockSpec((8, 128), lambda i: (0, 0)),
            pl.BlockSpec((8, 2 * D), lambda i: (0, 0)),
            pl.BlockSpec((8, 8), lambda i: (0, 0)),
        ],
        out_specs=pl.BlockSpec((B, 8), lambda i: (0, 0)),
        out_shape=jax.ShapeDtypeStruct((B, 8), _f32),
        scratch_shapes=[
            pltpu.VMEM((B, D), _f32),
            pltpu.VMEM((B, 1), _f32),
            pltpu.VMEM((B, D), _f32),
            pltpu.VMEM((B, 1), _f32),
        ],
    )(acc_r, y_r, deg_r, batch_r, acc_l, y_l, deg_l, batch_l, b2p, Wfc8, bfc8)


def _prep_nodes(x):
    xp = jnp.concatenate([x.astype(_i32), jnp.zeros((NP - N,), _i32)])
    return xp.reshape(NS, NCH_N, CH)


def _prep_edges(ei):
    pad = jnp.concatenate(
        [jnp.zeros((1, EP - E), _i32), jnp.full((1, EP - E), N, _i32)], axis=0)
    e = jnp.concatenate([ei.astype(_i32), pad], axis=1)
    return e.reshape(2, ECH, CHE)


def _prep_batch(b):
    bp = jnp.concatenate([b.astype(_i32), jnp.full((NP - N,), B + 7, _i32)])
    return bp.reshape(NBLK, PBLK, 8).transpose(0, 2, 1)


def kernel(receptor_x, receptor_edge_index, receptor_batch,
           ligand_x, ligand_edge_index, ligand_batch,
           emb_table, W1, b1, W2, b2, Wfc, bfc):
    x_r = _prep_nodes(receptor_x)
    x_l = _prep_nodes(ligand_x)
    e_r = _prep_edges(receptor_edge_index)
    e_l = _prep_edges(ligand_edge_index)
    batch_r = _prep_batch(receptor_batch)
    batch_l = _prep_batch(ligand_batch)

    zeros = jnp.zeros((NP, D), _f32)
    ones = jnp.ones((CHE, D), _f32)
    eye8 = jnp.eye(8, dtype=_f32)
    W1bd = jnp.kron(eye8, W1)
    W2bd = jnp.kron(eye8, W2)
    b1p = jnp.tile(jnp.tile(b1, 8)[None, :], (8, 1))
    b2p = jnp.tile(jnp.tile(b2, 8)[None, :], (8, 1))
    Wfc8 = jnp.concatenate([Wfc, jnp.zeros((2, 2 * D), _f32)], axis=0)
    bfc8 = jnp.tile(jnp.concatenate([bfc, jnp.zeros((2,), _f32)])[None, :],
                    (8, 1))

    def pack(a):
        return jnp.reshape(a, (NPR, 128))

    def unpack(a):
        return jnp.reshape(a, (NP, D))

    h0_r, h0_l, deg_r, deg_l = _build_sc_emb_deg()(
        x_r, x_l, e_r, e_l, emb_table, zeros, ones)
    h0_r, h0_l, deg_r, deg_l = map(pack, (h0_r, h0_l, deg_r, deg_l))

    y1_r = _tc_y1(h0_r, deg_r, W1bd)
    y1_l = _tc_y1(h0_l, deg_l, W1bd)

    acc1_r, acc1_l = _build_sc_aggregate()(
        unpack(y1_r), unpack(y1_l), e_r, e_l, zeros)
    acc1_r, acc1_l = pack(acc1_r), pack(acc1_l)

    y2_r = _tc_y2(acc1_r, y1_r, deg_r, W2bd, b1p)
    y2_l = _tc_y2(acc1_l, y1_l, deg_l, W2bd, b1p)

    acc2_r, acc2_l = _build_sc_aggregate()(
        unpack(y2_r), unpack(y2_l), e_r, e_l, zeros)
    acc2_r, acc2_l = pack(acc2_r), pack(acc2_l)

    out8 = _tc_epilogue(acc2_r, y2_r, deg_r, batch_r,
                        acc2_l, y2_l, deg_l, batch_l,
                        b2p, Wfc8, bfc8)
    return (out8[:, :3], out8[:, 3:6])


# speedup vs baseline: 98.1204x; 1.0030x over previous
import functools

import jax
import jax.numpy as jnp
from jax import lax
from jax.experimental import pallas as pl
from jax.experimental.pallas import tpu as pltpu
from jax.experimental.pallas import tpu_sc as plsc

N = 100000
E = 3200000
B = 128
D = 16
VOCAB = 1032

NS = 16
CH = 128
CHE = 256
NP = 100352
EP = 3276800
NPR = NP // 8
ECH = EP // CHE
NCH_N = NP // (NS * CH)
NCH_E = EP // (NS * CHE)
AGRP = 8
NBUF = 4
GLAG = 2
DLAG = 8
ROWS_PER_TILE = NP // NS
PBLK = 784
NBLK = NPR // PBLK

_f32 = jnp.float32
_i32 = jnp.int32

_SC_PARAMS = pltpu.CompilerParams(use_tc_tiling_on_sc=False)


@functools.cache
def _build_sc_emb_deg():
    @functools.partial(
        pl.kernel,
        out_type=(
            jax.ShapeDtypeStruct((NP, D), _f32),
            jax.ShapeDtypeStruct((NP, D), _f32),
            jax.ShapeDtypeStruct((NP, D), jnp.bfloat16),
            jax.ShapeDtypeStruct((NP, D), jnp.bfloat16),
        ),
        mesh=plsc.VectorSubcoreMesh(core_axis_name="c", subcore_axis_name="s"),
        compiler_params=_SC_PARAMS,
        scratch_types=[
            pltpu.VMEM_SHARED((NP, D), jnp.bfloat16),
            pltpu.VMEM((NCH_N, CH), _i32),
            pltpu.VMEM((3, AGRP * CHE), _i32),
            pltpu.VMEM((2, CH, D), _f32),
            pltpu.VMEM((CHE, D), jnp.bfloat16),
            pltpu.SemaphoreType.DMA((2,)),
            pltpu.SemaphoreType.DMA,
            pltpu.SemaphoreType.DMA,
        ],
    )
    def sc_emb_deg(x_r, x_l, e_r, e_l, emb, zeros, ones,
                   h0_r, h0_l, deg_r, deg_l,
                   acc, xbuf, ibuf, rows, ones_v, gsem, ssem, dsem):
        c = lax.axis_index("c")
        t = lax.axis_index("s")
        sl = pl.ds(t * ROWS_PER_TILE, ROWS_PER_TILE)
        ngrp = NCH_E // AGRP

        def side(x_hbm, e_hbm, h0_hbm, deg_hbm):
            pltpu.sync_copy(zeros.at[sl], acc.at[sl])
            pltpu.sync_copy(ones, ones_v)
            pltpu.sync_copy(x_hbm.at[t], xbuf)

            def h0_start(k, b):
                pltpu.make_async_copy(emb.at[xbuf.at[k]], rows.at[b],
                                      gsem.at[b]).start()

            def h0_wait(b):
                pltpu.make_async_copy(emb.at[xbuf.at[0]], rows.at[b],
                                      gsem.at[b]).wait()

            h0_start(0, 0)

            @pl.loop(0, NCH_N)
            def _(k):
                b = lax.rem(k, 2)

                @pl.when(k + 1 < NCH_N)
                def _():
                    h0_start(k + 1, 1 - b)

                h0_wait(b)
                pltpu.sync_copy(rows.at[b],
                                h0_hbm.at[pl.ds((t * NCH_N + k) * CH, CH)])

            plsc.subcore_barrier()

            def stage_start(g):
                base = EP + (t * NCH_E + g * AGRP) * CHE
                pltpu.make_async_copy(e_hbm.at[pl.ds(base, AGRP * CHE)],
                                      ibuf.at[lax.rem(g, 3)], ssem).start()

            def stage_wait(g):
                pltpu.make_async_copy(e_hbm.at[pl.ds(EP, AGRP * CHE)],
                                      ibuf.at[lax.rem(g, 3)], ssem).wait()

            def dscat_start(g, r):
                pltpu.make_async_copy(
                    ones_v,
                    acc.at[ibuf.at[lax.rem(g, 3), pl.ds(r * CHE, CHE)]],
                    dsem,
                ).start(add=True)

            def dscat_wait():
                pltpu.make_async_copy(ones_v, acc.at[ibuf.at[0, pl.ds(0, CHE)]],
                                      dsem).wait()

            stage_start(0)
            stage_wait(0)
            stage_start(1)

            @pl.loop(0, NCH_E)
            def _(i):
                g = lax.div(i, AGRP)
                r = lax.rem(i, AGRP)

                @pl.when((r == 0) & (g > 0))
                def _():
                    stage_wait(g)

                    @pl.when(g + 1 < ngrp)
                    def _():
                        stage_start(g + 1)

                dscat_start(g, r)

                @pl.when(i >= DLAG)
                def _():
                    dscat_wait()

            @pl.loop(0, DLAG)
            def _(i):
                dscat_wait()

            plsc.subcore_barrier()
            pltpu.sync_copy(acc.at[sl], deg_hbm.at[sl])

        @pl.when(c == 0)
        def _():
            side(x_r, e_r, h0_r, deg_r)

        @pl.when(c == 1)
        def _():
            side(x_l, e_l, h0_l, deg_l)

    return sc_emb_deg


@functools.cache
def _build_sc_aggregate():
    @functools.partial(
        pl.kernel,
        out_type=(
            jax.ShapeDtypeStruct((NP, D), _f32),
            jax.ShapeDtypeStruct((NP, D), _f32),
        ),
        mesh=plsc.VectorSubcoreMesh(core_axis_name="c", subcore_axis_name="s"),
        compiler_params=_SC_PARAMS,
        scratch_types=[
            pltpu.VMEM_SHARED((NP, D), _f32),
            pltpu.VMEM((3, AGRP * CHE), _i32),
            pltpu.VMEM((3, AGRP * CHE), _i32),
            pltpu.VMEM((NBUF, CHE, D), _f32),
            pltpu.SemaphoreType.DMA((NBUF,)),
            pltpu.SemaphoreType.DMA((NBUF,)),
            pltpu.SemaphoreType.DMA,
        ],
    )
    def sc_aggregate(y_r, y_l, e_r, e_l, zeros,
                     out_r, out_l,
                     acc, sbuf, dbuf, rows, gsem, ksem, ssem):
        c = lax.axis_index("c")
        t = lax.axis_index("s")
        sl = pl.ds(t * ROWS_PER_TILE, ROWS_PER_TILE)
        ngrp = NCH_E // AGRP
        m = NCH_E

        def side(y_hbm, e_hbm, out_hbm):
            pltpu.sync_copy(zeros.at[sl], acc.at[sl])
            plsc.subcore_barrier()

            def stage_start(g):
                base = (t * NCH_E + g * AGRP) * CHE
                slot = lax.rem(g, 3)
                pltpu.make_async_copy(e_hbm.at[pl.ds(base, AGRP * CHE)],
                                      sbuf.at[slot], ssem).start()
                pltpu.make_async_copy(e_hbm.at[pl.ds(EP + base, AGRP * CHE)],
                                      dbuf.at[slot], ssem).start()

            def stage_wait(g):
                slot = lax.rem(g, 3)
                pltpu.make_async_copy(e_hbm.at[pl.ds(0, AGRP * CHE)],
                                      sbuf.at[slot], ssem).wait()
                pltpu.make_async_copy(e_hbm.at[pl.ds(EP, AGRP * CHE)],
                                      dbuf.at[slot], ssem).wait()

            def gather_start(i, b):
                pltpu.make_async_copy(
                    y_hbm.at[sbuf.at[lax.rem(lax.div(i, AGRP), 3),
                                     pl.ds(lax.rem(i, AGRP) * CHE, CHE)]],
                    rows.at[b], gsem.at[b]).start()

            def gather_wait(b):
                pltpu.make_async_copy(y_hbm.at[sbuf.at[0, pl.ds(0, CHE)]],
                                      rows.at[b], gsem.at[b]).wait()

            def scat_start(i, b):
                pltpu.make_async_copy(
                    rows.at[b],
                    acc.at[dbuf.at[lax.rem(lax.div(i, AGRP), 3),
                                   pl.ds(lax.rem(i, AGRP) * CHE, CHE)]],
                    ksem.at[b]).start(add=True)

            def scat_wait(b):
                pltpu.make_async_copy(rows.at[b],
                                      acc.at[dbuf.at[0, pl.ds(0, CHE)]],
                                      ksem.at[b]).wait()

            stage_start(0)
            stage_wait(0)
            stage_start(1)

            @pl.loop(0, m)
            def _(i):
                g = lax.div(i, AGRP)
                r = lax.rem(i, AGRP)

                @pl.when((r == 0) & (g > 0))
                def _():
                    stage_wait(g)

                    @pl.when(g + 1 < ngrp)
                    def _():
                        stage_start(g + 1)

                b = lax.rem(i, NBUF)

                @pl.when(i >= NBUF)
                def _():
                    scat_wait(b)

                gather_start(i, b)

                @pl.when(i >= GLAG)
                def _():
                    b2 = lax.rem(i - GLAG, NBUF)
                    gather_wait(b2)
                    scat_start(i - GLAG, b2)

            @pl.loop(m, m + GLAG)
            def _(i):
                b2 = lax.rem(i - GLAG, NBUF)
                gather_wait(b2)
                scat_start(i - GLAG, b2)

            @pl.loop(0, NBUF)
            def _(b):
                scat_wait(b)

            plsc.subcore_barrier()
            pltpu.sync_copy(acc.at[sl], out_hbm.at[sl])

        @pl.when(c == 0)
        def _():
            side(y_r, e_r, out_r)

        @pl.when(c == 1)
        def _():
            side(y_l, e_l, out_l)

    return sc_aggregate


def _tc_y1(h0, deg, W1bd):
    def body(h_ref, d_ref, w_ref, y_ref):
        dinv = lax.rsqrt(d_ref[...] + 1.0)
        y_ref[...] = jnp.dot(h_ref[...], w_ref[...],
                             preferred_element_type=_f32) * dinv

    return pl.pallas_call(
        body,
        grid=(NBLK,),
        in_specs=[
            pl.BlockSpec((PBLK, 128), lambda i: (i, 0)),
            pl.BlockSpec((PBLK, 128), lambda i: (i, 0)),
            pl.BlockSpec((128, 128), lambda i: (0, 0)),
        ],
        out_specs=pl.BlockSpec((PBLK, 128), lambda i: (i, 0)),
        out_shape=jax.ShapeDtypeStruct((NPR, 128), _f32),
    )(h0, deg, W1bd)


def _tc_y2(acc1, y1, deg, W2bd, b1p):
    def body(a_ref, y_ref, d_ref, w_ref, b_ref, o_ref):
        dinv = lax.rsqrt(d_ref[...] + 1.0)
        h1 = (a_ref[...] + y_ref[...]) * dinv + b_ref[0:1, :]
        h1 = jnp.maximum(h1, 0.0)
        o_ref[...] = jnp.dot(h1, w_ref[...], preferred_element_type=_f32) * dinv

    return pl.pallas_call(
        body,
        grid=(NBLK,),
        in_specs=[
            pl.BlockSpec((PBLK, 128), lambda i: (i, 0)),
            pl.BlockSpec((PBLK, 128), lambda i: (i, 0)),
            pl.BlockSpec((PBLK, 128), lambda i: (i, 0)),
            pl.BlockSpec((128, 128), lambda i: (0, 0)),
            pl.BlockSpec((8, 128), lambda i: (0, 0)),
        ],
        out_specs=pl.BlockSpec((PBLK, 128), lambda i: (i, 0)),
        out_shape=jax.ShapeDtypeStruct((NPR, 128), _f32),
    )(acc1, y1, deg, W2bd, b1p)


def _tc_epilogue(acc_r, y_r, deg_r, batch_r, acc_l, y_l, deg_l, batch_l,
                 b2p, Wfc8, bfc8):
    nodes = PBLK * 8

    def body(ar, yr, dr, br, al, yl, dl, bl, b2_ref, w_ref, bf_ref, o_ref,
             s_r, c_r, s_l, c_l):
        i = pl.program_id(0)

        @pl.when(i == 0)
        def _():
            s_r[...] = jnp.zeros_like(s_r)
            c_r[...] = jnp.zeros_like(c_r)
            s_l[...] = jnp.zeros_like(s_l)
            c_l[...] = jnp.zeros_like(c_l)

        def side(a_ref, y_ref, d_ref, b_ref, s_scr, c_scr):
            dinv = lax.rsqrt(d_ref[...] + 1.0)
            h2p = (a_ref[...] + y_ref[...]) * dinv + b2_ref[0:1, :]
            bid8 = b_ref[0]
            s = jnp.zeros((B, D), _f32)
            cnt = jnp.zeros((B, 1), _f32)
            for k in range(8):
                row = bid8[k:k + 1, :]
                oh = (lax.broadcasted_iota(_i32, (B, PBLK), 0)
                      == row).astype(_f32)
                s = s + jnp.dot(oh, h2p[:, 16 * k:16 * (k + 1)],
                                preferred_element_type=_f32)
                cnt = cnt + jnp.sum(oh, axis=1, keepdims=True)
            s_scr[...] += s
            c_scr[...] += cnt

        side(ar, yr, dr, br, s_r, c_r)
        side(al, yl, dl, bl, s_l, c_l)

        @pl.when(i == NBLK - 1)
        def _():
            mr = s_r[...] / jnp.maximum(c_r[...], 1.0)
            ml = s_l[...] / jnp.maximum(c_l[...], 1.0)
            hcat = jnp.concatenate([mr, ml], axis=1)
            out = lax.dot_general(hcat, w_ref[...],
                                  (((1,), (1,)), ((), ())),
                                  preferred_element_type=_f32)
            o_ref[...] = out + bf_ref[0:1, :]

    node_spec = pl.BlockSpec((PBLK, 128), lambda i: (i, 0))
    batch_spec = pl.BlockSpec((1, 8, PBLK), lambda i: (i, 0, 0))
    return pl.pallas_call(
        body,
        grid=(NBLK,),
        in_specs=[
            node_spec, node_spec, node_spec, batch_spec,
            node_spec, node_spec, node_spec, batch_spec,
            pl.BlockSpec((8, 128), lambda i: (0, 0)),
            pl.BlockSpec((8, 2 * D), lambda i: (0, 0)),
            pl.BlockSpec((8, 8), lambda i: (0, 0)),
        ],
        out_specs=pl.BlockSpec((B, 8), lambda i: (0, 0)),
        out_shape=jax.ShapeDtypeStruct((B, 8), _f32),
        scratch_shapes=[
            pltpu.VMEM((B, D), _f32),
            pltpu.VMEM((B, 1), _f32),
            pltpu.VMEM((B, D), _f32),
            pltpu.VMEM((B, 1), _f32),
        ],
    )(acc_r, y_r, deg_r, batch_r, acc_l, y_l, deg_l, batch_l, b2p, Wfc8, bfc8)


def _prep_nodes(x):
    xp = jnp.concatenate([x.astype(_i32), jnp.zeros((NP - N,), _i32)])
    return xp.reshape(NS, NCH_N, CH)


def _prep_edges(ei):
    src = jnp.concatenate([ei[0].astype(_i32), jnp.zeros((EP - E,), _i32)])
    dst = jnp.concatenate([ei[1].astype(_i32), jnp.full((EP - E,), N, _i32)])
    return jnp.concatenate([src, dst])


def _prep_batch(b):
    bp = jnp.concatenate([b.astype(_i32), jnp.full((NP - N,), B + 7, _i32)])
    return bp.reshape(NBLK, PBLK, 8).transpose(0, 2, 1)


def kernel(receptor_x, receptor_edge_index, receptor_batch,
           ligand_x, ligand_edge_index, ligand_batch,
           emb_table, W1, b1, W2, b2, Wfc, bfc):
    x_r = _prep_nodes(receptor_x)
    x_l = _prep_nodes(ligand_x)
    e_r = _prep_edges(receptor_edge_index)
    e_l = _prep_edges(ligand_edge_index)
    batch_r = _prep_batch(receptor_batch)
    batch_l = _prep_batch(ligand_batch)

    zeros = jnp.zeros((NP, D), _f32)
    zeros_bf = jnp.zeros((NP, D), jnp.bfloat16)
    ones_bf = jnp.ones((CHE, D), jnp.bfloat16)
    eye8 = jnp.eye(8, dtype=_f32)
    W1bd = jnp.kron(eye8, W1)
    W2bd = jnp.kron(eye8, W2)
    b1p = jnp.tile(jnp.tile(b1, 8)[None, :], (8, 1))
    b2p = jnp.tile(jnp.tile(b2, 8)[None, :], (8, 1))
    Wfc8 = jnp.concatenate([Wfc, jnp.zeros((2, 2 * D), _f32)], axis=0)
    bfc8 = jnp.tile(jnp.concatenate([bfc, jnp.zeros((2,), _f32)])[None, :],
                    (8, 1))

    def pack(a):
        return jnp.reshape(a, (NPR, 128))

    def unpack(a):
        return jnp.reshape(a, (NP, D))

    h0_r, h0_l, deg_r, deg_l = _build_sc_emb_deg()(
        x_r, x_l, e_r, e_l, emb_table, zeros_bf, ones_bf)
    deg_r = deg_r.astype(_f32)
    deg_l = deg_l.astype(_f32)
    h0_r, h0_l, deg_r, deg_l = map(pack, (h0_r, h0_l, deg_r, deg_l))

    y1_r = _tc_y1(h0_r, deg_r, W1bd)
    y1_l = _tc_y1(h0_l, deg_l, W1bd)

    acc1_r, acc1_l = _build_sc_aggregate()(
        unpack(y1_r), unpack(y1_l), e_r, e_l, zeros)
    acc1_r, acc1_l = pack(acc1_r), pack(acc1_l)

    y2_r = _tc_y2(acc1_r, y1_r, deg_r, W2bd, b1p)
    y2_l = _tc_y2(acc1_l, y1_l, deg_l, W2bd, b1p)

    acc2_r, acc2_l = _build_sc_aggregate()(
        unpack(y2_r), unpack(y2_l), e_r, e_l, zeros)
    acc2_r, acc2_l = pack(acc2_r), pack(acc2_l)

    out8 = _tc_epilogue(acc2_r, y2_r, deg_r, batch_r,
                        acc2_l, y2_l, deg_l, batch_l,
                        b2p, Wfc8, bfc8)
    return (out8[:, :3], out8[:, 3:6])
